# Initial kernel scaffold; baseline (speedup 1.0000x reference)
import jax
import jax.numpy as jnp
from jax.experimental import pallas as pl


def kernel(h, pos, edge_index, edge_attr, params):
    raise NotImplementedError("write your pallas kernel here")


# speedup vs baseline: 2.5195x; 2.5195x over previous
import jax
import jax.numpy as jnp
from jax import lax
from jax.experimental import pallas as pl
from jax.experimental.pallas import tpu as pltpu
from jax.experimental.pallas import tpu_sc as plsc

N = 10000
E = 320000
H = 64
W = 128

_info = plsc.get_sparse_core_info()
NC = _info.num_cores
NS = _info.num_subcores
NW = NC * NS
EW = E // NW
C = 80
ITERS = EW // C
RPS = 624
RTAIL = N - (NS - 1) * RPS
EPC = E // NC

_f32 = jnp.float32


def _silu(x):
    return x / (1.0 + jnp.exp(-x))


def _sds(shape):
    return jax.ShapeDtypeStruct(shape, _f32)


_MESH = plsc.VectorSubcoreMesh(core_axis_name="c", subcore_axis_name="s")


def _gather_body(row_h, col_h, t1_h, t2_h, g1_h, g2_h,
                 idx_r, idx_c, buf_a, buf_b, sem):
    wid = lax.axis_index("s") * NC + lax.axis_index("c")

    def body(i, carry):
        base = wid * EW + i * C
        pltpu.sync_copy(row_h.at[pl.ds(base, C)], idx_r)
        pltpu.sync_copy(col_h.at[pl.ds(base, C)], idx_c)
        cp1 = pltpu.async_copy(t1_h.at[idx_r], buf_a, sem)
        cp2 = pltpu.async_copy(t2_h.at[idx_c], buf_b, sem)
        cp1.wait()
        cp2.wait()
        pltpu.sync_copy(buf_a, g1_h.at[pl.ds(base, C)])
        pltpu.sync_copy(buf_b, g2_h.at[pl.ds(base, C)])
        return carry

    lax.fori_loop(0, ITERS, body, 0)


_gather = pl.kernel(
    _gather_body,
    out_type=[_sds((E, W)), _sds((E, W))],
    mesh=_MESH,
    scratch_types=[
        pltpu.VMEM((C,), jnp.int32), pltpu.VMEM((C,), jnp.int32),
        pltpu.VMEM((C, W), _f32), pltpu.VMEM((C, W), _f32),
        pltpu.SemaphoreType.DMA,
    ],
)


def _scatter_body(row_h, data_h, zeros_h, out0_h, out1_h, idx_v, data_v,
                  shared):
    c = lax.axis_index("c")
    s = lax.axis_index("s")
    r0 = s * RPS
    last = NS - 1

    @pl.when(s != last)
    def _():
        pltpu.sync_copy(zeros_h.at[pl.ds(0, RPS)], shared.at[pl.ds(r0, RPS)])

    @pl.when(s == last)
    def _():
        pltpu.sync_copy(zeros_h, shared.at[pl.ds(last * RPS, RTAIL)])

    plsc.subcore_barrier()

    def it(i, carry):
        base = c * EPC + s * EW + i * C
        pltpu.sync_copy(row_h.at[pl.ds(base, C)], idx_v)
        pltpu.sync_copy(data_h.at[pl.ds(base, C)], data_v)
        pltpu.sync_copy(data_v, shared.at[idx_v], add=True)
        return carry

    lax.fori_loop(0, ITERS, it, 0)
    plsc.subcore_barrier()

    @pl.when((c == 0) & (s != last))
    def _():
        pltpu.sync_copy(shared.at[pl.ds(r0, RPS)], out0_h.at[pl.ds(r0, RPS)])

    @pl.when((c == 0) & (s == last))
    def _():
        pltpu.sync_copy(shared.at[pl.ds(last * RPS, RTAIL)],
                        out0_h.at[pl.ds(last * RPS, RTAIL)])

    @pl.when((c == 1) & (s != last))
    def _():
        pltpu.sync_copy(shared.at[pl.ds(r0, RPS)], out1_h.at[pl.ds(r0, RPS)])

    @pl.when((c == 1) & (s == last))
    def _():
        pltpu.sync_copy(shared.at[pl.ds(last * RPS, RTAIL)],
                        out1_h.at[pl.ds(last * RPS, RTAIL)])


_scatter = pl.kernel(
    _scatter_body,
    out_type=[_sds((N, W)), _sds((N, W))],
    mesh=_MESH,
    scratch_types=[
        pltpu.VMEM((C,), jnp.int32),
        pltpu.VMEM((C, W), _f32),
        pltpu.VMEM_SHARED((N, W), _f32),
    ],
)


BE = 2560
RN = 2000
_EGRID = E // BE
_NGRID = N // RN


def _b(shape):
    return pl.BlockSpec(shape, lambda i: tuple(0 for _ in shape))


def _r(rows, cols):
    return pl.BlockSpec((rows, cols), lambda i: (i, 0))


def _dot(a, b):
    return jnp.dot(a, b, preferred_element_type=_f32)


def _zeros(rows, cols):
    return jnp.zeros((rows, cols), _f32)


def _prep0_body(h_r, pos_r, embw_r, embb_r, w1a_r, b1_r, w1b_r,
                h64_o, t1_o, t2_o):
    h64 = _dot(h_r[...], embw_r[...]) + embb_r[...]
    h64_o[...] = h64
    pospad = jnp.concatenate([pos_r[...], _zeros(RN, 13)], axis=1)
    a = _dot(h64, w1a_r[...]) + b1_r[...]
    b = _dot(h64, w1b_r[...])
    t1_o[...] = jnp.concatenate([a, pospad, _zeros(RN, 48)], axis=1)
    t2_o[...] = jnp.concatenate([b, pospad, _zeros(RN, 48)], axis=1)


def _edge_geom(g1, g2):
    cd3 = g1[:, H:H + 3] - g2[:, H:H + 3]
    radial = jnp.sum(cd3 * cd3, axis=1, keepdims=True)
    cdiff = cd3 / (jnp.sqrt(radial + 1e-8) + 1.0)
    return radial, cdiff


def _edge1_body_l0(g1_r, g2_r, x_r, wr_r, wx_r, bx_r, w2_r, b2_r, m_o):
    g1 = g1_r[...]
    g2 = g2_r[...]
    radial, cdiff = _edge_geom(g1, g2)
    pre1 = (g1[:, :H] + g2[:, :H] + radial * wr_r[...]
            + _dot(x_r[...], wx_r[...]) + bx_r[...])
    mij = _silu(_dot(_silu(pre1), w2_r[...]) + b2_r[...])
    m_o[...] = jnp.concatenate([mij, cdiff, _zeros(BE, W - H - 3)], axis=1)


def _edge1_body_l1(g1_r, g2_r, x_r, wr_r, wx_r, w2_r, b2_r, wout_r, bout_r,
                   m_o):
    g1 = g1_r[...]
    g2 = g2_r[...]
    radial, cdiff = _edge_geom(g1, g2)
    pre1 = (g1[:, :H] + g2[:, :H] + radial * wr_r[...]
            + _dot(x_r[:, :H], wx_r[...]))
    mij = _silu(_dot(_silu(pre1), w2_r[...]) + b2_r[...])
    eaout = _dot(mij, wout_r[...]) + bout_r[...]
    m_o[...] = jnp.concatenate(
        [mij, cdiff, eaout, _zeros(BE, W - H - 3 - 8)], axis=1)


def _edge2_body(g1_r, g2_r, m_r, wm_r, w2_r, b2_r, w3_r, t_o):
    m = m_r[...]
    pre = g1_r[:, :H] + g2_r[:, H:] + _dot(m[:, :H], wm_r[...])
    u = _silu(_dot(_silu(pre), w2_r[...]) + b2_r[...])
    phi = _dot(u, w3_r[...])[:, 0:1]
    trans = m[:, H:H + 3] * phi
    t_o[...] = jnp.concatenate([trans, _zeros(BE, W - 3)], axis=1)


def _node_body(h_r, p0_r, p1_r, w1a_r, w1b_r, b1_r, w2_r, b2_r,
               cw1a_r, cw1b_r, cb1_r, hn_o, ab_o):
    agg = (p0_r[:, :H] + p1_r[:, :H]) * 0.01
    pre = _dot(h_r[...], w1a_r[...]) + _dot(agg, w1b_r[...]) + b1_r[...]
    hn = h_r[...] + _dot(_silu(pre), w2_r[...]) + b2_r[...]
    hn_o[...] = hn
    aeq = _dot(hn, cw1a_r[...]) + cb1_r[...]
    beq = _dot(hn, cw1b_r[...])
    ab_o[...] = jnp.concatenate([aeq, beq], axis=1)


def _posprep_body(pp_r, q0_r, q1_r, h_r, w1a_r, b1_r, w1b_r,
                  pp_o, t1_o, t2_o):
    pp1 = pp_r[...] + (q0_r[:, :16] + q1_r[:, :16]) * 0.01
    pp_o[...] = pp1
    a = _dot(h_r[...], w1a_r[...]) + b1_r[...]
    b = _dot(h_r[...], w1b_r[...])
    t1_o[...] = jnp.concatenate([a, pp1, _zeros(RN, 48)], axis=1)
    t2_o[...] = jnp.concatenate([b, pp1, _zeros(RN, 48)], axis=1)


def _final_body(pp_r, q0_r, q1_r, h_r, ew_r, eb_r, pos_o, hout_o):
    pos_o[...] = pp_r[...] + (q0_r[:, :16] + q1_r[:, :16]) * 0.01
    hout_o[...] = _dot(h_r[...], ew_r[...]) + eb_r[...]


def _prep0(h, pos, embw, embb, w1a, b1, w1b):
    return pl.pallas_call(
        _prep0_body,
        grid=(_NGRID,),
        in_specs=[_r(RN, 8), _r(RN, 3), _b((8, H)), _b((1, H)),
                  _b((H, H)), _b((1, H)), _b((H, H))],
        out_specs=[_r(RN, H), _r(RN, W), _r(RN, W)],
        out_shape=[_sds((N, H)), _sds((N, W)), _sds((N, W))],
    )(h, pos, embw, embb, w1a, b1, w1b)


def _edge1_l0(g1, g2, x, wr, wx, bx, w2, b2):
    return pl.pallas_call(
        _edge1_body_l0,
        grid=(_EGRID,),
        in_specs=[_r(BE, W), _r(BE, W), _r(BE, 2),
                  _b((1, H)), _b((2, H)), _b((1, H)), _b((H, H)), _b((1, H))],
        out_specs=[_r(BE, W)],
        out_shape=[_sds((E, W))],
    )(g1, g2, x, wr, wx, bx, w2, b2)[0]


def _edge1_l1(g1, g2, x, wr, wx, w2, b2, wout, bout):
    return pl.pallas_call(
        _edge1_body_l1,
        grid=(_EGRID,),
        in_specs=[_r(BE, W), _r(BE, W), _r(BE, W),
                  _b((1, H)), _b((H, H)), _b((H, H)), _b((1, H)),
                  _b((H, 8)), _b((1, 8))],
        out_specs=[_r(BE, W)],
        out_shape=[_sds((E, W))],
    )(g1, g2, x, wr, wx, w2, b2, wout, bout)[0]


def _edge2(g1, g2, m, wm, w2, b2, w3):
    return pl.pallas_call(
        _edge2_body,
        grid=(_EGRID,),
        in_specs=[_r(BE, W), _r(BE, W), _r(BE, W),
                  _b((H, H)), _b((H, H)), _b((1, H)), _b((H, 8))],
        out_specs=[_r(BE, W)],
        out_shape=[_sds((E, W))],
    )(g1, g2, m, wm, w2, b2, w3)[0]


def _node(h64, p0, p1, w1a, w1b, b1, w2, b2, cw1a, cw1b, cb1):
    return pl.pallas_call(
        _node_body,
        grid=(_NGRID,),
        in_specs=[_r(RN, H), _r(RN, W), _r(RN, W),
                  _b((H, H)), _b((H, H)), _b((1, H)), _b((H, H)), _b((1, H)),
                  _b((H, H)), _b((H, H)), _b((1, H))],
        out_specs=[_r(RN, H), _r(RN, W)],
        out_shape=[_sds((N, H)), _sds((N, W))],
    )(h64, p0, p1, w1a, w1b, b1, w2, b2, cw1a, cw1b, cb1)


def _posprep(pp, q0, q1, h1, w1a, b1, w1b):
    return pl.pallas_call(
        _posprep_body,
        grid=(_NGRID,),
        in_specs=[_r(RN, 16), _r(RN, W), _r(RN, W), _r(RN, H),
                  _b((H, H)), _b((1, H)), _b((H, H))],
        out_specs=[_r(RN, 16), _r(RN, W), _r(RN, W)],
        out_shape=[_sds((N, 16)), _sds((N, W)), _sds((N, W))],
    )(pp, q0, q1, h1, w1a, b1, w1b)


def _final(pp, q0, q1, h2, ew, eb):
    return pl.pallas_call(
        _final_body,
        grid=(_NGRID,),
        in_specs=[_r(RN, 16), _r(RN, W), _r(RN, W), _r(RN, H),
                  _b((H, 8)), _b((1, 8))],
        out_specs=[_r(RN, 16), _r(RN, 8)],
        out_shape=[_sds((N, 16)), _sds((N, 8))],
    )(pp, q0, q1, h2, ew, eb)


def kernel(h, pos, edge_index, edge_attr, params):
    p = params
    row = edge_index[0]
    col = edge_index[1]

    g0 = p['block_0']['gcl_0']
    e0 = p['block_0']['equiv']
    g1 = p['block_1']['gcl_0']
    e1 = p['block_1']['equiv']

    def rsh(v):
        return v.reshape(1, -1)

    w1a0, w1b0, w1c0 = g0['e_w1'][:H], g0['e_w1'][H:2 * H], g0['e_w1'][2 * H:]
    w1a1, w1b1, w1c1 = g1['e_w1'][:H], g1['e_w1'][H:2 * H], g1['e_w1'][2 * H:]
    wr0 = rsh(w1c0[0] + p['eemb_w'][0] @ w1c0[1:])
    wx0 = p['eemb_w'][1:3] @ w1c0[1:]
    bx0 = rsh(p['eemb_b'] @ w1c0[1:])
    wr1 = rsh(w1c1[0])
    wx1 = w1c1.at[0].set(0.0)
    wout = jnp.concatenate(
        [jnp.zeros((1, 3), _f32), p['eemb_out_w']], axis=0)
    wout = jnp.concatenate([wout, jnp.zeros((H, 5), _f32)], axis=1)
    bout = jnp.concatenate(
        [p['eemb_out_b'], jnp.zeros((5,), _f32)]).reshape(1, 8)
    cw1a0, cw1b0, cw1c0 = e0['c_w1'][:H], e0['c_w1'][H:2 * H], e0['c_w1'][2 * H:]
    cw1a1, cw1b1, cw1c1 = e1['c_w1'][:H], e1['c_w1'][H:2 * H], e1['c_w1'][2 * H:]
    w3_0 = jnp.concatenate([e0['c_w3'], jnp.zeros((H, 7), _f32)], axis=1)
    w3_1 = jnp.concatenate([e1['c_w3'], jnp.zeros((H, 7), _f32)], axis=1)

    z_w = jnp.zeros((RTAIL, W), _f32)
    pp0 = jnp.concatenate([pos, jnp.zeros((N, 13), _f32)], axis=1)

    h64, t1, t2 = _prep0(h, pos, p['emb_w'], rsh(p['emb_b']),
                         w1a0, rsh(g0['e_b1']), w1b0)

    gr1, gr2 = _gather(row, col, t1, t2)
    m0 = _edge1_l0(gr1, gr2, edge_attr, wr0, wx0, bx0,
                   g0['e_w2'], rsh(g0['e_b2']))
    s0, s1 = _scatter(row, m0, z_w)
    h1, ab = _node(h64, s0, s1, g0['n_w1'][:H], g0['n_w1'][H:],
                   rsh(g0['n_b1']), g0['n_w2'], rsh(g0['n_b2']),
                   cw1a0, cw1b0, rsh(e0['c_b1']))
    ge1, ge2 = _gather(row, col, ab, ab)
    tr0 = _edge2(ge1, ge2, m0, cw1c0, e0['c_w2'], rsh(e0['c_b2']), w3_0)
    q0, q1 = _scatter(row, tr0, z_w)
    pp1, t1, t2 = _posprep(pp0, q0, q1, h1, w1a1, rsh(g1['e_b1']), w1b1)

    gr1, gr2 = _gather(row, col, t1, t2)
    m1 = _edge1_l1(gr1, gr2, m0, wr1, wx1, g1['e_w2'], rsh(g1['e_b2']),
                   wout, bout)
    s0, s1 = _scatter(row, m1, z_w)
    h2, ab = _node(h1, s0, s1, g1['n_w1'][:H], g1['n_w1'][H:],
                   rsh(g1['n_b1']), g1['n_w2'], rsh(g1['n_b2']),
                   cw1a1, cw1b1, rsh(e1['c_b1']))
    ge1, ge2 = _gather(row, col, ab, ab)
    tr1 = _edge2(ge1, ge2, m1, cw1c1, e1['c_w2'], rsh(e1['c_b2']), w3_1)
    q0, q1 = _scatter(row, tr1, z_w)
    pos_pad, h_out = _final(pp1, q0, q1, h2, p['emb_out_w'],
                            rsh(p['emb_out_b']))

    return h_out, pos_pad[:, :3], m1[:, H + 3:H + 6]


# speedup vs baseline: 3.4772x; 1.3801x over previous
import jax
import jax.numpy as jnp
from jax import lax
from jax.experimental import pallas as pl
from jax.experimental.pallas import tpu as pltpu
from jax.experimental.pallas import tpu_sc as plsc

N = 10000
E = 320000
H = 64
W = 128

_info = plsc.get_sparse_core_info()
NC = _info.num_cores
NS = _info.num_subcores
NW = NC * NS
EW = E // NW
C = 80
K = 5
CH = C * K
ITERS = EW // CH
RPS = 624
RTAIL = N - (NS - 1) * RPS
EPC = E // NC

_f32 = jnp.float32


def _silu(x):
    return x / (1.0 + jnp.exp(-x))


def _sds(shape):
    return jax.ShapeDtypeStruct(shape, _f32)


_MESH = plsc.VectorSubcoreMesh(core_axis_name="c", subcore_axis_name="s")


def _gather_body(row_h, col_h, t1_h, t2_h, g1_h, g2_h,
                 idx_r, idx_c, buf_a, buf_b, sga, sgb, swa, swb):
    wid = lax.axis_index("s") * NC + lax.axis_index("c")
    tb = wid * EW

    def ga(j):
        return pltpu.make_async_copy(
            t1_h.at[idx_r.at[pl.ds(j * C, C)]], buf_a.at[j], sga)

    def gb(j):
        return pltpu.make_async_copy(
            t2_h.at[idx_c.at[pl.ds(j * C, C)]], buf_b.at[j], sgb)

    def wa(base, j):
        return pltpu.make_async_copy(
            buf_a.at[j], g1_h.at[pl.ds(base + j * C, C)], swa)

    def wb(base, j):
        return pltpu.make_async_copy(
            buf_b.at[j], g2_h.at[pl.ds(base + j * C, C)], swb)

    def body(g, carry):
        base = tb + g * CH
        for j in range(K):
            ga(j).wait()
        for j in range(K):
            wa(base, j).start()

        @pl.when(g > 0)
        def _():
            for j in range(K):
                wb(tb, j).wait()

        for j in range(K):
            gb(j).start()
        for j in range(K):
            gb(j).wait()
        for j in range(K):
            wb(base, j).start()

        @pl.when(g < ITERS - 1)
        def _():
            pltpu.sync_copy(row_h.at[pl.ds(base + CH, CH)], idx_r)
            pltpu.sync_copy(col_h.at[pl.ds(base + CH, CH)], idx_c)
            for j in range(K):
                wa(tb, j).wait()
            for j in range(K):
                ga(j).start()

        return carry

    pltpu.sync_copy(row_h.at[pl.ds(tb, CH)], idx_r)
    pltpu.sync_copy(col_h.at[pl.ds(tb, CH)], idx_c)
    for j in range(K):
        ga(j).start()
    lax.fori_loop(0, ITERS, body, 0)
    for j in range(K):
        wa(tb, j).wait()
        wb(tb, j).wait()


_gather = pl.kernel(
    _gather_body,
    out_type=[_sds((E, W)), _sds((E, W))],
    mesh=_MESH,
    scratch_types=[
        pltpu.VMEM((CH,), jnp.int32), pltpu.VMEM((CH,), jnp.int32),
        pltpu.VMEM((K, C, W), _f32), pltpu.VMEM((K, C, W), _f32),
        pltpu.SemaphoreType.DMA, pltpu.SemaphoreType.DMA,
        pltpu.SemaphoreType.DMA, pltpu.SemaphoreType.DMA,
    ],
)


def _scatter_body(row_h, data_h, zeros_h, out0_h, out1_h,
                  idx_a, idx_b, dat_a, dat_b, shared, sa, sb):
    c = lax.axis_index("c")
    s = lax.axis_index("s")
    r0 = s * RPS
    last = NS - 1
    tb = c * EPC + s * EW

    def fire(buf_idx, buf_dat, sem, base):
        pltpu.make_async_copy(
            row_h.at[pl.ds(base, C)], buf_idx, sem).start()
        pltpu.make_async_copy(
            data_h.at[pl.ds(base, C)], buf_dat, sem).start()

    def drain(buf_idx, buf_dat, sem):
        pltpu.make_async_copy(
            row_h.at[pl.ds(tb, C)], buf_idx, sem).wait()
        pltpu.make_async_copy(
            data_h.at[pl.ds(tb, C)], buf_dat, sem).wait()

    def adds(buf_idx, buf_dat):
        pltpu.sync_copy(buf_dat, shared.at[buf_idx], add=True)

    @pl.when(s != last)
    def _():
        pltpu.sync_copy(zeros_h.at[pl.ds(0, RPS)], shared.at[pl.ds(r0, RPS)])

    @pl.when(s == last)
    def _():
        pltpu.sync_copy(zeros_h, shared.at[pl.ds(last * RPS, RTAIL)])

    plsc.subcore_barrier()

    fire(idx_a, dat_a, sa, tb)

    def body(g, carry):
        b0 = tb + 2 * g * C
        drain(idx_a, dat_a, sa)
        fire(idx_b, dat_b, sb, b0 + C)
        adds(idx_a, dat_a)
        drain(idx_b, dat_b, sb)
        fire(idx_a, dat_a, sa, b0 + 2 * C)
        adds(idx_b, dat_b)
        return carry

    lax.fori_loop(0, (EW // C - 1) // 2, body, 0)
    drain(idx_a, dat_a, sa)
    adds(idx_a, dat_a)
    plsc.subcore_barrier()

    @pl.when((c == 0) & (s != last))
    def _():
        pltpu.sync_copy(shared.at[pl.ds(r0, RPS)], out0_h.at[pl.ds(r0, RPS)])

    @pl.when((c == 0) & (s == last))
    def _():
        pltpu.sync_copy(shared.at[pl.ds(last * RPS, RTAIL)],
                        out0_h.at[pl.ds(last * RPS, RTAIL)])

    @pl.when((c == 1) & (s != last))
    def _():
        pltpu.sync_copy(shared.at[pl.ds(r0, RPS)], out1_h.at[pl.ds(r0, RPS)])

    @pl.when((c == 1) & (s == last))
    def _():
        pltpu.sync_copy(shared.at[pl.ds(last * RPS, RTAIL)],
                        out1_h.at[pl.ds(last * RPS, RTAIL)])


_scatter = pl.kernel(
    _scatter_body,
    out_type=[_sds((N, W)), _sds((N, W))],
    mesh=_MESH,
    scratch_types=[
        pltpu.VMEM((C,), jnp.int32), pltpu.VMEM((C,), jnp.int32),
        pltpu.VMEM((C, W), _f32), pltpu.VMEM((C, W), _f32),
        pltpu.VMEM_SHARED((N, W), _f32),
        pltpu.SemaphoreType.DMA, pltpu.SemaphoreType.DMA,
    ],
)


BE = 2560
RN = 2000
_EGRID = E // BE
_NGRID = N // RN


def _b(shape):
    return pl.BlockSpec(shape, lambda i: tuple(0 for _ in shape))


def _r(rows, cols):
    return pl.BlockSpec((rows, cols), lambda i: (i, 0))


def _dot(a, b):
    return jnp.dot(a, b, preferred_element_type=_f32)


def _zeros(rows, cols):
    return jnp.zeros((rows, cols), _f32)


def _prep0_body(h_r, pos_r, embw_r, embb_r, w1a_r, b1_r, w1b_r,
                h64_o, t1_o, t2_o):
    h64 = _dot(h_r[...], embw_r[...]) + embb_r[...]
    h64_o[...] = h64
    pospad = jnp.concatenate([pos_r[...], _zeros(RN, 13)], axis=1)
    a = _dot(h64, w1a_r[...]) + b1_r[...]
    b = _dot(h64, w1b_r[...])
    t1_o[...] = jnp.concatenate([a, pospad, _zeros(RN, 48)], axis=1)
    t2_o[...] = jnp.concatenate([b, pospad, _zeros(RN, 48)], axis=1)


def _edge_geom(g1, g2):
    cd3 = g1[:, H:H + 3] - g2[:, H:H + 3]
    radial = jnp.sum(cd3 * cd3, axis=1, keepdims=True)
    cdiff = cd3 / (jnp.sqrt(radial + 1e-8) + 1.0)
    return radial, cdiff


def _edge1_body_l0(g1_r, g2_r, x_r, wr_r, wx_r, bx_r, w2_r, b2_r, m_o):
    g1 = g1_r[...]
    g2 = g2_r[...]
    radial, cdiff = _edge_geom(g1, g2)
    pre1 = (g1[:, :H] + g2[:, :H] + radial * wr_r[...]
            + _dot(x_r[...], wx_r[...]) + bx_r[...])
    mij = _silu(_dot(_silu(pre1), w2_r[...]) + b2_r[...])
    m_o[...] = jnp.concatenate([mij, cdiff, _zeros(BE, W - H - 3)], axis=1)


def _edge1_body_l1(g1_r, g2_r, x_r, wr_r, wx_r, w2_r, b2_r, wout_r, bout_r,
                   m_o):
    g1 = g1_r[...]
    g2 = g2_r[...]
    radial, cdiff = _edge_geom(g1, g2)
    pre1 = (g1[:, :H] + g2[:, :H] + radial * wr_r[...]
            + _dot(x_r[:, :H], wx_r[...]))
    mij = _silu(_dot(_silu(pre1), w2_r[...]) + b2_r[...])
    eaout = _dot(mij, wout_r[...]) + bout_r[...]
    m_o[...] = jnp.concatenate(
        [mij, cdiff, eaout, _zeros(BE, W - H - 3 - 8)], axis=1)


def _edge2_body(g1_r, g2_r, m_r, wm_r, w2_r, b2_r, w3_r, t_o):
    m = m_r[...]
    pre = g1_r[:, :H] + g2_r[:, H:] + _dot(m[:, :H], wm_r[...])
    u = _silu(_dot(_silu(pre), w2_r[...]) + b2_r[...])
    phi = _dot(u, w3_r[...])[:, 0:1]
    trans = m[:, H:H + 3] * phi
    t_o[...] = jnp.concatenate([trans, _zeros(BE, W - 3)], axis=1)


def _node_body(h_r, p0_r, p1_r, w1a_r, w1b_r, b1_r, w2_r, b2_r,
               cw1a_r, cw1b_r, cb1_r, hn_o, ab_o):
    agg = (p0_r[:, :H] + p1_r[:, :H]) * 0.01
    pre = _dot(h_r[...], w1a_r[...]) + _dot(agg, w1b_r[...]) + b1_r[...]
    hn = h_r[...] + _dot(_silu(pre), w2_r[...]) + b2_r[...]
    hn_o[...] = hn
    aeq = _dot(hn, cw1a_r[...]) + cb1_r[...]
    beq = _dot(hn, cw1b_r[...])
    ab_o[...] = jnp.concatenate([aeq, beq], axis=1)


def _posprep_body(pp_r, q0_r, q1_r, h_r, w1a_r, b1_r, w1b_r,
                  pp_o, t1_o, t2_o):
    pp1 = pp_r[...] + (q0_r[:, :16] + q1_r[:, :16]) * 0.01
    pp_o[...] = pp1
    a = _dot(h_r[...], w1a_r[...]) + b1_r[...]
    b = _dot(h_r[...], w1b_r[...])
    t1_o[...] = jnp.concatenate([a, pp1, _zeros(RN, 48)], axis=1)
    t2_o[...] = jnp.concatenate([b, pp1, _zeros(RN, 48)], axis=1)


def _final_body(pp_r, q0_r, q1_r, h_r, ew_r, eb_r, pos_o, hout_o):
    pos_o[...] = pp_r[...] + (q0_r[:, :16] + q1_r[:, :16]) * 0.01
    hout_o[...] = _dot(h_r[...], ew_r[...]) + eb_r[...]


def _prep0(h, pos, embw, embb, w1a, b1, w1b):
    return pl.pallas_call(
        _prep0_body,
        grid=(_NGRID,),
        in_specs=[_r(RN, 8), _r(RN, 3), _b((8, H)), _b((1, H)),
                  _b((H, H)), _b((1, H)), _b((H, H))],
        out_specs=[_r(RN, H), _r(RN, W), _r(RN, W)],
        out_shape=[_sds((N, H)), _sds((N, W)), _sds((N, W))],
    )(h, pos, embw, embb, w1a, b1, w1b)


def _edge1_l0(g1, g2, x, wr, wx, bx, w2, b2):
    return pl.pallas_call(
        _edge1_body_l0,
        grid=(_EGRID,),
        in_specs=[_r(BE, W), _r(BE, W), _r(BE, 2),
                  _b((1, H)), _b((2, H)), _b((1, H)), _b((H, H)), _b((1, H))],
        out_specs=[_r(BE, W)],
        out_shape=[_sds((E, W))],
    )(g1, g2, x, wr, wx, bx, w2, b2)[0]


def _edge1_l1(g1, g2, x, wr, wx, w2, b2, wout, bout):
    return pl.pallas_call(
        _edge1_body_l1,
        grid=(_EGRID,),
        in_specs=[_r(BE, W), _r(BE, W), _r(BE, W),
                  _b((1, H)), _b((H, H)), _b((H, H)), _b((1, H)),
                  _b((H, 8)), _b((1, 8))],
        out_specs=[_r(BE, W)],
        out_shape=[_sds((E, W))],
    )(g1, g2, x, wr, wx, w2, b2, wout, bout)[0]


def _edge2(g1, g2, m, wm, w2, b2, w3):
    return pl.pallas_call(
        _edge2_body,
        grid=(_EGRID,),
        in_specs=[_r(BE, W), _r(BE, W), _r(BE, W),
                  _b((H, H)), _b((H, H)), _b((1, H)), _b((H, 8))],
        out_specs=[_r(BE, W)],
        out_shape=[_sds((E, W))],
    )(g1, g2, m, wm, w2, b2, w3)[0]


def _node(h64, p0, p1, w1a, w1b, b1, w2, b2, cw1a, cw1b, cb1):
    return pl.pallas_call(
        _node_body,
        grid=(_NGRID,),
        in_specs=[_r(RN, H), _r(RN, W), _r(RN, W),
                  _b((H, H)), _b((H, H)), _b((1, H)), _b((H, H)), _b((1, H)),
                  _b((H, H)), _b((H, H)), _b((1, H))],
        out_specs=[_r(RN, H), _r(RN, W)],
        out_shape=[_sds((N, H)), _sds((N, W))],
    )(h64, p0, p1, w1a, w1b, b1, w2, b2, cw1a, cw1b, cb1)


def _posprep(pp, q0, q1, h1, w1a, b1, w1b):
    return pl.pallas_call(
        _posprep_body,
        grid=(_NGRID,),
        in_specs=[_r(RN, 16), _r(RN, W), _r(RN, W), _r(RN, H),
                  _b((H, H)), _b((1, H)), _b((H, H))],
        out_specs=[_r(RN, 16), _r(RN, W), _r(RN, W)],
        out_shape=[_sds((N, 16)), _sds((N, W)), _sds((N, W))],
    )(pp, q0, q1, h1, w1a, b1, w1b)


def _final(pp, q0, q1, h2, ew, eb):
    return pl.pallas_call(
        _final_body,
        grid=(_NGRID,),
        in_specs=[_r(RN, 16), _r(RN, W), _r(RN, W), _r(RN, H),
                  _b((H, 8)), _b((1, 8))],
        out_specs=[_r(RN, 16), _r(RN, 8)],
        out_shape=[_sds((N, 16)), _sds((N, 8))],
    )(pp, q0, q1, h2, ew, eb)


def kernel(h, pos, edge_index, edge_attr, params):
    p = params
    row = edge_index[0]
    col = edge_index[1]

    g0 = p['block_0']['gcl_0']
    e0 = p['block_0']['equiv']
    g1 = p['block_1']['gcl_0']
    e1 = p['block_1']['equiv']

    def rsh(v):
        return v.reshape(1, -1)

    w1a0, w1b0, w1c0 = g0['e_w1'][:H], g0['e_w1'][H:2 * H], g0['e_w1'][2 * H:]
    w1a1, w1b1, w1c1 = g1['e_w1'][:H], g1['e_w1'][H:2 * H], g1['e_w1'][2 * H:]
    wr0 = rsh(w1c0[0] + p['eemb_w'][0] @ w1c0[1:])
    wx0 = p['eemb_w'][1:3] @ w1c0[1:]
    bx0 = rsh(p['eemb_b'] @ w1c0[1:])
    wr1 = rsh(w1c1[0])
    wx1 = w1c1.at[0].set(0.0)
    wout = jnp.concatenate(
        [jnp.zeros((1, 3), _f32), p['eemb_out_w']], axis=0)
    wout = jnp.concatenate([wout, jnp.zeros((H, 5), _f32)], axis=1)
    bout = jnp.concatenate(
        [p['eemb_out_b'], jnp.zeros((5,), _f32)]).reshape(1, 8)
    cw1a0, cw1b0, cw1c0 = e0['c_w1'][:H], e0['c_w1'][H:2 * H], e0['c_w1'][2 * H:]
    cw1a1, cw1b1, cw1c1 = e1['c_w1'][:H], e1['c_w1'][H:2 * H], e1['c_w1'][2 * H:]
    w3_0 = jnp.concatenate([e0['c_w3'], jnp.zeros((H, 7), _f32)], axis=1)
    w3_1 = jnp.concatenate([e1['c_w3'], jnp.zeros((H, 7), _f32)], axis=1)

    z_w = jnp.zeros((RTAIL, W), _f32)
    pp0 = jnp.concatenate([pos, jnp.zeros((N, 13), _f32)], axis=1)

    h64, t1, t2 = _prep0(h, pos, p['emb_w'], rsh(p['emb_b']),
                         w1a0, rsh(g0['e_b1']), w1b0)

    gr1, gr2 = _gather(row, col, t1, t2)
    m0 = _edge1_l0(gr1, gr2, edge_attr, wr0, wx0, bx0,
                   g0['e_w2'], rsh(g0['e_b2']))
    s0, s1 = _scatter(row, m0, z_w)
    h1, ab = _node(h64, s0, s1, g0['n_w1'][:H], g0['n_w1'][H:],
                   rsh(g0['n_b1']), g0['n_w2'], rsh(g0['n_b2']),
                   cw1a0, cw1b0, rsh(e0['c_b1']))
    ge1, ge2 = _gather(row, col, ab, ab)
    tr0 = _edge2(ge1, ge2, m0, cw1c0, e0['c_w2'], rsh(e0['c_b2']), w3_0)
    q0, q1 = _scatter(row, tr0, z_w)
    pp1, t1, t2 = _posprep(pp0, q0, q1, h1, w1a1, rsh(g1['e_b1']), w1b1)

    gr1, gr2 = _gather(row, col, t1, t2)
    m1 = _edge1_l1(gr1, gr2, m0, wr1, wx1, g1['e_w2'], rsh(g1['e_b2']),
                   wout, bout)
    s0, s1 = _scatter(row, m1, z_w)
    h2, ab = _node(h1, s0, s1, g1['n_w1'][:H], g1['n_w1'][H:],
                   rsh(g1['n_b1']), g1['n_w2'], rsh(g1['n_b2']),
                   cw1a1, cw1b1, rsh(e1['c_b1']))
    ge1, ge2 = _gather(row, col, ab, ab)
    tr1 = _edge2(ge1, ge2, m1, cw1c1, e1['c_w2'], rsh(e1['c_b2']), w3_1)
    q0, q1 = _scatter(row, tr1, z_w)
    pos_pad, h_out = _final(pp1, q0, q1, h2, p['emb_out_w'],
                            rsh(p['emb_out_b']))

    return h_out, pos_pad[:, :3], m1[:, H + 3:H + 6]


# speedup vs baseline: 3.5755x; 1.0283x over previous
import jax
import jax.numpy as jnp
from jax import lax
from jax.experimental import pallas as pl
from jax.experimental.pallas import tpu as pltpu
from jax.experimental.pallas import tpu_sc as plsc

N = 10000
E = 320000
H = 64
W = 128

_info = plsc.get_sparse_core_info()
NC = _info.num_cores
NS = _info.num_subcores
NW = NC * NS
EW = E // NW
C = 80
K = 5
CH = C * K
ITERS = EW // CH
RPS = 624
RTAIL = N - (NS - 1) * RPS
EPC = E // NC

_f32 = jnp.float32
_bf16 = jnp.bfloat16


def _silu(x):
    return x / (1.0 + jnp.exp(-x))


def _sds(shape, dtype=jnp.float32):
    return jax.ShapeDtypeStruct(shape, dtype)


_MESH = plsc.VectorSubcoreMesh(core_axis_name="c", subcore_axis_name="s")


def _gather_body(row_h, col_h, t1_h, t2_h, g1_h, g2_h,
                 idx_r, idx_c, buf_a, buf_b, sga, sgb, swa, swb):
    wid = lax.axis_index("s") * NC + lax.axis_index("c")
    tb = wid * EW

    def ga(j):
        return pltpu.make_async_copy(
            t1_h.at[idx_r.at[pl.ds(j * C, C)]], buf_a.at[j], sga)

    def gb(j):
        return pltpu.make_async_copy(
            t2_h.at[idx_c.at[pl.ds(j * C, C)]], buf_b.at[j], sgb)

    def wa(base, j):
        return pltpu.make_async_copy(
            buf_a.at[j], g1_h.at[pl.ds(base + j * C, C)], swa)

    def wb(base, j):
        return pltpu.make_async_copy(
            buf_b.at[j], g2_h.at[pl.ds(base + j * C, C)], swb)

    def body(g, carry):
        base = tb + g * CH
        for j in range(K):
            ga(j).wait()
        for j in range(K):
            wa(base, j).start()

        @pl.when(g > 0)
        def _():
            for j in range(K):
                wb(tb, j).wait()

        for j in range(K):
            gb(j).start()
        for j in range(K):
            gb(j).wait()
        for j in range(K):
            wb(base, j).start()

        @pl.when(g < ITERS - 1)
        def _():
            pltpu.sync_copy(row_h.at[pl.ds(base + CH, CH)], idx_r)
            pltpu.sync_copy(col_h.at[pl.ds(base + CH, CH)], idx_c)
            for j in range(K):
                wa(tb, j).wait()
            for j in range(K):
                ga(j).start()

        return carry

    pltpu.sync_copy(row_h.at[pl.ds(tb, CH)], idx_r)
    pltpu.sync_copy(col_h.at[pl.ds(tb, CH)], idx_c)
    for j in range(K):
        ga(j).start()
    lax.fori_loop(0, ITERS, body, 0)
    for j in range(K):
        wa(tb, j).wait()
        wb(tb, j).wait()


_gather = pl.kernel(
    _gather_body,
    out_type=[_sds((E, W)), _sds((E, W))],
    mesh=_MESH,
    scratch_types=[
        pltpu.VMEM((CH,), jnp.int32), pltpu.VMEM((CH,), jnp.int32),
        pltpu.VMEM((K, C, W), _f32), pltpu.VMEM((K, C, W), _f32),
        pltpu.SemaphoreType.DMA, pltpu.SemaphoreType.DMA,
        pltpu.SemaphoreType.DMA, pltpu.SemaphoreType.DMA,
    ],
)


def _scatter_body(row_h, data_h, zeros_h, out0_h, out1_h,
                  idx_a, idx_b, dat_a, dat_b, shared, sa, sb):
    c = lax.axis_index("c")
    s = lax.axis_index("s")
    r0 = s * RPS
    last = NS - 1
    tb = c * EPC + s * EW

    def fire(buf_idx, buf_dat, sem, base):
        pltpu.make_async_copy(
            row_h.at[pl.ds(base, C)], buf_idx, sem).start()
        pltpu.make_async_copy(
            data_h.at[pl.ds(base, C)], buf_dat, sem).start()

    def drain(buf_idx, buf_dat, sem):
        pltpu.make_async_copy(
            row_h.at[pl.ds(tb, C)], buf_idx, sem).wait()
        pltpu.make_async_copy(
            data_h.at[pl.ds(tb, C)], buf_dat, sem).wait()

    def adds(buf_idx, buf_dat):
        pltpu.sync_copy(buf_dat, shared.at[buf_idx], add=True)

    @pl.when(s != last)
    def _():
        pltpu.sync_copy(zeros_h.at[pl.ds(0, RPS)], shared.at[pl.ds(r0, RPS)])

    @pl.when(s == last)
    def _():
        pltpu.sync_copy(zeros_h, shared.at[pl.ds(last * RPS, RTAIL)])

    plsc.subcore_barrier()

    fire(idx_a, dat_a, sa, tb)

    def body(g, carry):
        b0 = tb + 2 * g * C
        drain(idx_a, dat_a, sa)
        fire(idx_b, dat_b, sb, b0 + C)
        adds(idx_a, dat_a)
        drain(idx_b, dat_b, sb)
        fire(idx_a, dat_a, sa, b0 + 2 * C)
        adds(idx_b, dat_b)
        return carry

    lax.fori_loop(0, (EW // C - 1) // 2, body, 0)
    drain(idx_a, dat_a, sa)
    adds(idx_a, dat_a)
    plsc.subcore_barrier()

    @pl.when((c == 0) & (s != last))
    def _():
        pltpu.sync_copy(shared.at[pl.ds(r0, RPS)], out0_h.at[pl.ds(r0, RPS)])

    @pl.when((c == 0) & (s == last))
    def _():
        pltpu.sync_copy(shared.at[pl.ds(last * RPS, RTAIL)],
                        out0_h.at[pl.ds(last * RPS, RTAIL)])

    @pl.when((c == 1) & (s != last))
    def _():
        pltpu.sync_copy(shared.at[pl.ds(r0, RPS)], out1_h.at[pl.ds(r0, RPS)])

    @pl.when((c == 1) & (s == last))
    def _():
        pltpu.sync_copy(shared.at[pl.ds(last * RPS, RTAIL)],
                        out1_h.at[pl.ds(last * RPS, RTAIL)])


_scatter = pl.kernel(
    _scatter_body,
    out_type=[_sds((N, W)), _sds((N, W))],
    mesh=_MESH,
    scratch_types=[
        pltpu.VMEM((C,), jnp.int32), pltpu.VMEM((C,), jnp.int32),
        pltpu.VMEM((C, W), _f32), pltpu.VMEM((C, W), _f32),
        pltpu.VMEM_SHARED((N, W), _f32),
        pltpu.SemaphoreType.DMA, pltpu.SemaphoreType.DMA,
    ],
)


BE = 2560
RN = 2000
_EGRID = E // BE
_NGRID = N // RN


def _b(shape):
    return pl.BlockSpec(shape, lambda i: tuple(0 for _ in shape))


def _r(rows, cols):
    return pl.BlockSpec((rows, cols), lambda i: (i, 0))


def _dot(a, b):
    return jnp.dot(a, b, preferred_element_type=_f32)


def _bdot(a, b):
    return jnp.dot(a.astype(_bf16), b.astype(_bf16),
                   preferred_element_type=_f32)


def _zeros(rows, cols):
    return jnp.zeros((rows, cols), _f32)


def _prep0_body(h_r, pos_r, embw_r, embb_r, w1a_r, b1_r, w1b_r,
                h64_o, t1_o, t2_o):
    h64 = _dot(h_r[...], embw_r[...]) + embb_r[...]
    h64_o[...] = h64
    pospad = jnp.concatenate([pos_r[...], _zeros(RN, 13)], axis=1)
    a = _dot(h64, w1a_r[...]) + b1_r[...]
    b = _dot(h64, w1b_r[...])
    t1_o[...] = jnp.concatenate([a, pospad, _zeros(RN, 48)], axis=1)
    t2_o[...] = jnp.concatenate([b, pospad, _zeros(RN, 48)], axis=1)


def _edge_geom(g1, g2):
    cd3 = g1[:, H:H + 3] - g2[:, H:H + 3]
    radial = jnp.sum(cd3 * cd3, axis=1, keepdims=True)
    cdiff = cd3 / (jnp.sqrt(radial + 1e-8) + 1.0)
    return radial, cdiff


def _edge1_body_l0(g1_r, g2_r, x_r, wr_r, wx_r, bx_r, w2_r, b2_r, m_o):
    g1 = g1_r[...]
    g2 = g2_r[...]
    radial, cdiff = _edge_geom(g1, g2)
    pre1 = (g1[:, :H] + g2[:, :H] + radial * wr_r[...]
            + _bdot(x_r[...], wx_r[...]) + bx_r[...])
    mij = _silu(_bdot(_silu(pre1), w2_r[...]) + b2_r[...])
    m_o[...] = jnp.concatenate([mij, cdiff, _zeros(BE, W - H - 3)], axis=1)


def _edge1_body_l1(g1_r, g2_r, x_r, wr_r, wx_r, w2_r, b2_r, wout_r, bout_r,
                   m_o):
    g1 = g1_r[...]
    g2 = g2_r[...]
    radial, cdiff = _edge_geom(g1, g2)
    pre1 = (g1[:, :H] + g2[:, :H] + radial * wr_r[...]
            + _bdot(x_r[:, :H], wx_r[...]))
    mij = _silu(_bdot(_silu(pre1), w2_r[...]) + b2_r[...])
    eaout = _bdot(mij, wout_r[...]) + bout_r[...]
    m_o[...] = jnp.concatenate(
        [mij, cdiff, eaout, _zeros(BE, W - H - 3 - 8)], axis=1)


def _edge2_body(g1_r, g2_r, m_r, wm_r, w2_r, b2_r, w3_r, t_o):
    m = m_r[...]
    pre = g1_r[:, :H] + g2_r[:, H:] + _bdot(m[:, :H], wm_r[...])
    u = _silu(_bdot(_silu(pre), w2_r[...]) + b2_r[...])
    phi = _bdot(u, w3_r[...])[:, 0:1]
    trans = m[:, H:H + 3] * phi
    t_o[...] = jnp.concatenate([trans, _zeros(BE, W - 3)], axis=1)


def _node_body(h_r, p0_r, p1_r, w1a_r, w1b_r, b1_r, w2_r, b2_r,
               cw1a_r, cw1b_r, cb1_r, hn_o, ab_o):
    agg = (p0_r[:, :H] + p1_r[:, :H]) * 0.01
    pre = _dot(h_r[...], w1a_r[...]) + _dot(agg, w1b_r[...]) + b1_r[...]
    hn = h_r[...] + _dot(_silu(pre), w2_r[...]) + b2_r[...]
    hn_o[...] = hn
    aeq = _dot(hn, cw1a_r[...]) + cb1_r[...]
    beq = _dot(hn, cw1b_r[...])
    ab_o[...] = jnp.concatenate([aeq, beq], axis=1)


def _posprep_body(pp_r, q0_r, q1_r, h_r, w1a_r, b1_r, w1b_r,
                  pp_o, t1_o, t2_o):
    pp1 = pp_r[...] + (q0_r[:, :16] + q1_r[:, :16]) * 0.01
    pp_o[...] = pp1
    a = _dot(h_r[...], w1a_r[...]) + b1_r[...]
    b = _dot(h_r[...], w1b_r[...])
    t1_o[...] = jnp.concatenate([a, pp1, _zeros(RN, 48)], axis=1)
    t2_o[...] = jnp.concatenate([b, pp1, _zeros(RN, 48)], axis=1)


def _final_body(pp_r, q0_r, q1_r, h_r, ew_r, eb_r, pos_o, hout_o):
    pos_o[...] = pp_r[...] + (q0_r[:, :16] + q1_r[:, :16]) * 0.01
    hout_o[...] = _dot(h_r[...], ew_r[...]) + eb_r[...]


def _prep0(h, pos, embw, embb, w1a, b1, w1b):
    return pl.pallas_call(
        _prep0_body,
        grid=(_NGRID,),
        in_specs=[_r(RN, 8), _r(RN, 3), _b((8, H)), _b((1, H)),
                  _b((H, H)), _b((1, H)), _b((H, H))],
        out_specs=[_r(RN, H), _r(RN, W), _r(RN, W)],
        out_shape=[_sds((N, H)), _sds((N, W)), _sds((N, W))],
    )(h, pos, embw, embb, w1a, b1, w1b)


def _edge1_l0(g1, g2, x, wr, wx, bx, w2, b2):
    return pl.pallas_call(
        _edge1_body_l0,
        grid=(_EGRID,),
        in_specs=[_r(BE, W), _r(BE, W), _r(BE, 2),
                  _b((1, H)), _b((2, H)), _b((1, H)), _b((H, H)), _b((1, H))],
        out_specs=[_r(BE, W)],
        out_shape=[_sds((E, W))],
    )(g1, g2, x, wr, wx, bx, w2, b2)[0]


def _edge1_l1(g1, g2, x, wr, wx, w2, b2, wout, bout):
    return pl.pallas_call(
        _edge1_body_l1,
        grid=(_EGRID,),
        in_specs=[_r(BE, W), _r(BE, W), _r(BE, W),
                  _b((1, H)), _b((H, H)), _b((H, H)), _b((1, H)),
                  _b((H, 8)), _b((1, 8))],
        out_specs=[_r(BE, W)],
        out_shape=[_sds((E, W))],
    )(g1, g2, x, wr, wx, w2, b2, wout, bout)[0]


def _edge2(g1, g2, m, wm, w2, b2, w3):
    return pl.pallas_call(
        _edge2_body,
        grid=(_EGRID,),
        in_specs=[_r(BE, W), _r(BE, W), _r(BE, W),
                  _b((H, H)), _b((H, H)), _b((1, H)), _b((H, 8))],
        out_specs=[_r(BE, W)],
        out_shape=[_sds((E, W))],
    )(g1, g2, m, wm, w2, b2, w3)[0]


def _node(h64, p0, p1, w1a, w1b, b1, w2, b2, cw1a, cw1b, cb1):
    return pl.pallas_call(
        _node_body,
        grid=(_NGRID,),
        in_specs=[_r(RN, H), _r(RN, W), _r(RN, W),
                  _b((H, H)), _b((H, H)), _b((1, H)), _b((H, H)), _b((1, H)),
                  _b((H, H)), _b((H, H)), _b((1, H))],
        out_specs=[_r(RN, H), _r(RN, W)],
        out_shape=[_sds((N, H)), _sds((N, W))],
    )(h64, p0, p1, w1a, w1b, b1, w2, b2, cw1a, cw1b, cb1)


def _posprep(pp, q0, q1, h1, w1a, b1, w1b):
    return pl.pallas_call(
        _posprep_body,
        grid=(_NGRID,),
        in_specs=[_r(RN, 16), _r(RN, W), _r(RN, W), _r(RN, H),
                  _b((H, H)), _b((1, H)), _b((H, H))],
        out_specs=[_r(RN, 16), _r(RN, W), _r(RN, W)],
        out_shape=[_sds((N, 16)), _sds((N, W)), _sds((N, W))],
    )(pp, q0, q1, h1, w1a, b1, w1b)


def _final(pp, q0, q1, h2, ew, eb):
    return pl.pallas_call(
        _final_body,
        grid=(_NGRID,),
        in_specs=[_r(RN, 16), _r(RN, W), _r(RN, W), _r(RN, H),
                  _b((H, 8)), _b((1, 8))],
        out_specs=[_r(RN, 16), _r(RN, 8)],
        out_shape=[_sds((N, 16)), _sds((N, 8))],
    )(pp, q0, q1, h2, ew, eb)


def kernel(h, pos, edge_index, edge_attr, params):
    p = params
    row = edge_index[0]
    col = edge_index[1]

    g0 = p['block_0']['gcl_0']
    e0 = p['block_0']['equiv']
    g1 = p['block_1']['gcl_0']
    e1 = p['block_1']['equiv']

    def rsh(v):
        return v.reshape(1, -1)

    w1a0, w1b0, w1c0 = g0['e_w1'][:H], g0['e_w1'][H:2 * H], g0['e_w1'][2 * H:]
    w1a1, w1b1, w1c1 = g1['e_w1'][:H], g1['e_w1'][H:2 * H], g1['e_w1'][2 * H:]
    wr0 = rsh(w1c0[0] + p['eemb_w'][0] @ w1c0[1:])
    wx0 = p['eemb_w'][1:3] @ w1c0[1:]
    bx0 = rsh(p['eemb_b'] @ w1c0[1:])
    wr1 = rsh(w1c1[0])
    wx1 = w1c1.at[0].set(0.0)
    wout = jnp.concatenate(
        [jnp.zeros((1, 3), _f32), p['eemb_out_w']], axis=0)
    wout = jnp.concatenate([wout, jnp.zeros((H, 5), _f32)], axis=1)
    bout = jnp.concatenate(
        [p['eemb_out_b'], jnp.zeros((5,), _f32)]).reshape(1, 8)
    cw1a0, cw1b0, cw1c0 = e0['c_w1'][:H], e0['c_w1'][H:2 * H], e0['c_w1'][2 * H:]
    cw1a1, cw1b1, cw1c1 = e1['c_w1'][:H], e1['c_w1'][H:2 * H], e1['c_w1'][2 * H:]
    w3_0 = jnp.concatenate([e0['c_w3'], jnp.zeros((H, 7), _f32)], axis=1)
    w3_1 = jnp.concatenate([e1['c_w3'], jnp.zeros((H, 7), _f32)], axis=1)

    z_w = jnp.zeros((RTAIL, W), _f32)
    pp0 = jnp.concatenate([pos, jnp.zeros((N, 13), _f32)], axis=1)

    h64, t1, t2 = _prep0(h, pos, p['emb_w'], rsh(p['emb_b']),
                         w1a0, rsh(g0['e_b1']), w1b0)

    gr1, gr2 = _gather(row, col, t1, t2)
    m0 = _edge1_l0(gr1, gr2, edge_attr, wr0, wx0, bx0,
                   g0['e_w2'], rsh(g0['e_b2']))
    s0, s1 = _scatter(row, m0, z_w)
    h1, ab = _node(h64, s0, s1, g0['n_w1'][:H], g0['n_w1'][H:],
                   rsh(g0['n_b1']), g0['n_w2'], rsh(g0['n_b2']),
                   cw1a0, cw1b0, rsh(e0['c_b1']))
    ge1, ge2 = _gather(row, col, ab, ab)
    tr0 = _edge2(ge1, ge2, m0, cw1c0, e0['c_w2'], rsh(e0['c_b2']), w3_0)
    q0, q1 = _scatter(row, tr0, z_w)
    pp1, t1, t2 = _posprep(pp0, q0, q1, h1, w1a1, rsh(g1['e_b1']), w1b1)

    gr1, gr2 = _gather(row, col, t1, t2)
    m1 = _edge1_l1(gr1, gr2, m0, wr1, wx1, g1['e_w2'], rsh(g1['e_b2']),
                   wout, bout)
    s0, s1 = _scatter(row, m1, z_w)
    h2, ab = _node(h1, s0, s1, g1['n_w1'][:H], g1['n_w1'][H:],
                   rsh(g1['n_b1']), g1['n_w2'], rsh(g1['n_b2']),
                   cw1a1, cw1b1, rsh(e1['c_b1']))
    ge1, ge2 = _gather(row, col, ab, ab)
    tr1 = _edge2(ge1, ge2, m1, cw1c1, e1['c_w2'], rsh(e1['c_b2']), w3_1)
    q0, q1 = _scatter(row, tr1, z_w)
    pos_pad, h_out = _final(pp1, q0, q1, h2, p['emb_out_w'],
                            rsh(p['emb_out_b']))

    return h_out, pos_pad[:, :3], m1[:, H + 3:H + 6]


# speedup vs baseline: 3.6686x; 1.0260x over previous
import jax
import jax.numpy as jnp
from jax import lax
from jax.experimental import pallas as pl
from jax.experimental.pallas import tpu as pltpu
from jax.experimental.pallas import tpu_sc as plsc

N = 10000
E = 320000
H = 64
W = 128

_info = plsc.get_sparse_core_info()
NC = _info.num_cores
NS = _info.num_subcores
NW = NC * NS
EH = E // 2
EW = EH // NW
C = 40
K = 5
CH = C * K
ITERS = EW // CH
RPS = 624
RTAIL = N - (NS - 1) * RPS
EPC = EH // NC

_f32 = jnp.float32
_bf16 = jnp.bfloat16


def _silu(x):
    return x / (1.0 + jnp.exp(-x))


def _sds(shape, dtype=jnp.float32):
    return jax.ShapeDtypeStruct(shape, dtype)


_MESH = plsc.VectorSubcoreMesh(core_axis_name="c", subcore_axis_name="s")


def _gather_body(row_h, col_h, t1_h, t2_h, g1_h, g2_h,
                 idx_r, idx_c, buf_a, buf_b, sga, sgb, swa, swb):
    wid = lax.axis_index("s") * NC + lax.axis_index("c")
    tb = wid * EW

    def ga(j):
        return pltpu.make_async_copy(
            t1_h.at[idx_r.at[pl.ds(j * C, C)]], buf_a.at[j], sga)

    def gb(j):
        return pltpu.make_async_copy(
            t2_h.at[idx_c.at[pl.ds(j * C, C)]], buf_b.at[j], sgb)

    def wa(base, j):
        return pltpu.make_async_copy(
            buf_a.at[j], g1_h.at[pl.ds(base + j * C, C)], swa)

    def wb(base, j):
        return pltpu.make_async_copy(
            buf_b.at[j], g2_h.at[pl.ds(base + j * C, C)], swb)

    def body(g, carry):
        base = tb + g * CH
        for j in range(K):
            ga(j).wait()
        for j in range(K):
            wa(base, j).start()

        @pl.when(g > 0)
        def _():
            for j in range(K):
                wb(tb, j).wait()

        for j in range(K):
            gb(j).start()
        for j in range(K):
            gb(j).wait()
        for j in range(K):
            wb(base, j).start()

        @pl.when(g < ITERS - 1)
        def _():
            pltpu.sync_copy(row_h.at[pl.ds(base + CH, CH)], idx_r)
            pltpu.sync_copy(col_h.at[pl.ds(base + CH, CH)], idx_c)
            for j in range(K):
                wa(tb, j).wait()
            for j in range(K):
                ga(j).start()

        return carry

    pltpu.sync_copy(row_h.at[pl.ds(tb, CH)], idx_r)
    pltpu.sync_copy(col_h.at[pl.ds(tb, CH)], idx_c)
    for j in range(K):
        ga(j).start()
    lax.fori_loop(0, ITERS, body, 0)
    for j in range(K):
        wa(tb, j).wait()
        wb(tb, j).wait()


_gather = pl.kernel(
    _gather_body,
    out_type=[_sds((EH, W)), _sds((EH, W))],
    mesh=_MESH,
    scratch_types=[
        pltpu.VMEM((CH,), jnp.int32), pltpu.VMEM((CH,), jnp.int32),
        pltpu.VMEM((K, C, W), _f32), pltpu.VMEM((K, C, W), _f32),
        pltpu.SemaphoreType.DMA, pltpu.SemaphoreType.DMA,
        pltpu.SemaphoreType.DMA, pltpu.SemaphoreType.DMA,
    ],
)


def _scatter_body(row_h, data_h, zeros_h, out0_h, out1_h,
                  idx_a, idx_b, dat_a, dat_b, shared, sa, sb):
    c = lax.axis_index("c")
    s = lax.axis_index("s")
    r0 = s * RPS
    last = NS - 1
    tb = c * EPC + s * EW

    def fire(buf_idx, buf_dat, sem, base):
        pltpu.make_async_copy(
            row_h.at[pl.ds(base, C)], buf_idx, sem).start()
        pltpu.make_async_copy(
            data_h.at[pl.ds(base, C)], buf_dat, sem).start()

    def drain(buf_idx, buf_dat, sem):
        pltpu.make_async_copy(
            row_h.at[pl.ds(tb, C)], buf_idx, sem).wait()
        pltpu.make_async_copy(
            data_h.at[pl.ds(tb, C)], buf_dat, sem).wait()

    def adds(buf_idx, buf_dat):
        pltpu.sync_copy(buf_dat, shared.at[buf_idx], add=True)

    @pl.when(s != last)
    def _():
        pltpu.sync_copy(zeros_h.at[pl.ds(0, RPS)], shared.at[pl.ds(r0, RPS)])

    @pl.when(s == last)
    def _():
        pltpu.sync_copy(zeros_h, shared.at[pl.ds(last * RPS, RTAIL)])

    plsc.subcore_barrier()

    fire(idx_a, dat_a, sa, tb)

    def body(g, carry):
        b0 = tb + 2 * g * C
        drain(idx_a, dat_a, sa)
        fire(idx_b, dat_b, sb, b0 + C)
        adds(idx_a, dat_a)
        drain(idx_b, dat_b, sb)
        fire(idx_a, dat_a, sa, b0 + 2 * C)
        adds(idx_b, dat_b)
        return carry

    lax.fori_loop(0, (EW // C - 1) // 2, body, 0)
    drain(idx_a, dat_a, sa)
    adds(idx_a, dat_a)
    plsc.subcore_barrier()

    @pl.when((c == 0) & (s != last))
    def _():
        pltpu.sync_copy(shared.at[pl.ds(r0, RPS)], out0_h.at[pl.ds(r0, RPS)])

    @pl.when((c == 0) & (s == last))
    def _():
        pltpu.sync_copy(shared.at[pl.ds(last * RPS, RTAIL)],
                        out0_h.at[pl.ds(last * RPS, RTAIL)])

    @pl.when((c == 1) & (s != last))
    def _():
        pltpu.sync_copy(shared.at[pl.ds(r0, RPS)], out1_h.at[pl.ds(r0, RPS)])

    @pl.when((c == 1) & (s == last))
    def _():
        pltpu.sync_copy(shared.at[pl.ds(last * RPS, RTAIL)],
                        out1_h.at[pl.ds(last * RPS, RTAIL)])


_scatter = pl.kernel(
    _scatter_body,
    out_type=[_sds((N, W)), _sds((N, W))],
    mesh=_MESH,
    scratch_types=[
        pltpu.VMEM((C,), jnp.int32), pltpu.VMEM((C,), jnp.int32),
        pltpu.VMEM((C, W), _f32), pltpu.VMEM((C, W), _f32),
        pltpu.VMEM_SHARED((N, W), _f32),
        pltpu.SemaphoreType.DMA, pltpu.SemaphoreType.DMA,
    ],
)


BE = 3200
RN = 2000
_EGRID = EH // BE
_NGRID = N // RN


def _b(shape):
    return pl.BlockSpec(shape, lambda i: tuple(0 for _ in shape))


def _r(rows, cols):
    return pl.BlockSpec((rows, cols), lambda i: (i, 0))


def _dot(a, b):
    return jnp.dot(a, b, preferred_element_type=_f32)


def _bdot(a, b):
    return jnp.dot(a.astype(_bf16), b.astype(_bf16),
                   preferred_element_type=_f32)


def _zeros(rows, cols):
    return jnp.zeros((rows, cols), _f32)


def _prep0_body(h_r, pos_r, embw_r, embb_r, w1a_r, b1_r, w1b_r,
                h64_o, t1_o, t2_o):
    h64 = _dot(h_r[...], embw_r[...]) + embb_r[...]
    h64_o[...] = h64
    pospad = jnp.concatenate([pos_r[...], _zeros(RN, 13)], axis=1)
    a = _dot(h64, w1a_r[...]) + b1_r[...]
    b = _dot(h64, w1b_r[...])
    t1_o[...] = jnp.concatenate([a, pospad, _zeros(RN, 48)], axis=1)
    t2_o[...] = jnp.concatenate([b, pospad, _zeros(RN, 48)], axis=1)


def _edge_geom(g1, g2):
    cd3 = g1[:, H:H + 3] - g2[:, H:H + 3]
    radial = jnp.sum(cd3 * cd3, axis=1, keepdims=True)
    cdiff = cd3 / (jnp.sqrt(radial + 1e-8) + 1.0)
    return radial, cdiff


def _edge1_body_l0(g1_r, g2_r, x_r, wr_r, wx_r, bx_r, w2_r, b2_r, m_o):
    g1 = g1_r[...]
    g2 = g2_r[...]
    radial, cdiff = _edge_geom(g1, g2)
    pre1 = (g1[:, :H] + g2[:, :H] + radial * wr_r[...]
            + _bdot(x_r[...], wx_r[...]) + bx_r[...])
    mij = _silu(_bdot(_silu(pre1), w2_r[...]) + b2_r[...])
    m_o[...] = jnp.concatenate([mij, cdiff, _zeros(BE, W - H - 3)], axis=1)


def _edge1_body_l1(g1_r, g2_r, x_r, wr_r, wx_r, w2_r, b2_r, wout_r, bout_r,
                   m_o):
    g1 = g1_r[...]
    g2 = g2_r[...]
    radial, cdiff = _edge_geom(g1, g2)
    pre1 = (g1[:, :H] + g2[:, :H] + radial * wr_r[...]
            + _bdot(x_r[:, :H], wx_r[...]))
    mij = _silu(_bdot(_silu(pre1), w2_r[...]) + b2_r[...])
    eaout = _bdot(mij, wout_r[...]) + bout_r[...]
    m_o[...] = jnp.concatenate(
        [mij, cdiff, eaout, _zeros(BE, W - H - 3 - 8)], axis=1)


def _edge2_body(g1_r, g2_r, m_r, wm_r, w2_r, b2_r, w3_r, t_o):
    m = m_r[...]
    pre = g1_r[:, :H] + g2_r[:, H:] + _bdot(m[:, :H], wm_r[...])
    u = _silu(_bdot(_silu(pre), w2_r[...]) + b2_r[...])
    phi = _bdot(u, w3_r[...])[:, 0:1]
    trans = m[:, H:H + 3] * phi
    t_o[...] = jnp.concatenate([trans, _zeros(BE, W - 3)], axis=1)


def _node_body(h_r, p0_r, p1_r, p2_r, p3_r, w1a_r, w1b_r, b1_r, w2_r, b2_r,
               cw1a_r, cw1b_r, cb1_r, hn_o, ab_o):
    agg = (p0_r[:, :H] + p1_r[:, :H] + p2_r[:, :H] + p3_r[:, :H]) * 0.01
    pre = _dot(h_r[...], w1a_r[...]) + _dot(agg, w1b_r[...]) + b1_r[...]
    hn = h_r[...] + _dot(_silu(pre), w2_r[...]) + b2_r[...]
    hn_o[...] = hn
    aeq = _dot(hn, cw1a_r[...]) + cb1_r[...]
    beq = _dot(hn, cw1b_r[...])
    ab_o[...] = jnp.concatenate([aeq, beq], axis=1)


def _posprep_body(pp_r, q0_r, q1_r, q2_r, q3_r, h_r, w1a_r, b1_r, w1b_r,
                  pp_o, t1_o, t2_o):
    pp1 = pp_r[...] + (q0_r[:, :16] + q1_r[:, :16]
                       + q2_r[:, :16] + q3_r[:, :16]) * 0.01
    pp_o[...] = pp1
    a = _dot(h_r[...], w1a_r[...]) + b1_r[...]
    b = _dot(h_r[...], w1b_r[...])
    t1_o[...] = jnp.concatenate([a, pp1, _zeros(RN, 48)], axis=1)
    t2_o[...] = jnp.concatenate([b, pp1, _zeros(RN, 48)], axis=1)


def _final_body(pp_r, q0_r, q1_r, q2_r, q3_r, h_r, ew_r, eb_r,
                pos_o, hout_o):
    pos_o[...] = pp_r[...] + (q0_r[:, :16] + q1_r[:, :16]
                              + q2_r[:, :16] + q3_r[:, :16]) * 0.01
    hout_o[...] = _dot(h_r[...], ew_r[...]) + eb_r[...]


def _prep0(h, pos, embw, embb, w1a, b1, w1b):
    return pl.pallas_call(
        _prep0_body,
        grid=(_NGRID,),
        in_specs=[_r(RN, 8), _r(RN, 3), _b((8, H)), _b((1, H)),
                  _b((H, H)), _b((1, H)), _b((H, H))],
        out_specs=[_r(RN, H), _r(RN, W), _r(RN, W)],
        out_shape=[_sds((N, H)), _sds((N, W)), _sds((N, W))],
    )(h, pos, embw, embb, w1a, b1, w1b)


def _edge1_l0(g1, g2, x, wr, wx, bx, w2, b2):
    return pl.pallas_call(
        _edge1_body_l0,
        grid=(_EGRID,),
        in_specs=[_r(BE, W), _r(BE, W), _r(BE, 2),
                  _b((1, H)), _b((2, H)), _b((1, H)), _b((H, H)), _b((1, H))],
        out_specs=[_r(BE, W)],
        out_shape=[_sds((EH, W))],
    )(g1, g2, x, wr, wx, bx, w2, b2)[0]


def _edge1_l1(g1, g2, x, wr, wx, w2, b2, wout, bout):
    return pl.pallas_call(
        _edge1_body_l1,
        grid=(_EGRID,),
        in_specs=[_r(BE, W), _r(BE, W), _r(BE, W),
                  _b((1, H)), _b((H, H)), _b((H, H)), _b((1, H)),
                  _b((H, 8)), _b((1, 8))],
        out_specs=[_r(BE, W)],
        out_shape=[_sds((EH, W))],
    )(g1, g2, x, wr, wx, w2, b2, wout, bout)[0]


def _edge2(g1, g2, m, wm, w2, b2, w3):
    return pl.pallas_call(
        _edge2_body,
        grid=(_EGRID,),
        in_specs=[_r(BE, W), _r(BE, W), _r(BE, W),
                  _b((H, H)), _b((H, H)), _b((1, H)), _b((H, 8))],
        out_specs=[_r(BE, W)],
        out_shape=[_sds((EH, W))],
    )(g1, g2, m, wm, w2, b2, w3)[0]


def _node(h64, ps, w1a, w1b, b1, w2, b2, cw1a, cw1b, cb1):
    return pl.pallas_call(
        _node_body,
        grid=(_NGRID,),
        in_specs=[_r(RN, H)] + [_r(RN, W)] * 4 +
                 [_b((H, H)), _b((H, H)), _b((1, H)), _b((H, H)), _b((1, H)),
                  _b((H, H)), _b((H, H)), _b((1, H))],
        out_specs=[_r(RN, H), _r(RN, W)],
        out_shape=[_sds((N, H)), _sds((N, W))],
    )(h64, *ps, w1a, w1b, b1, w2, b2, cw1a, cw1b, cb1)


def _posprep(pp, qs, h1, w1a, b1, w1b):
    return pl.pallas_call(
        _posprep_body,
        grid=(_NGRID,),
        in_specs=[_r(RN, 16)] + [_r(RN, W)] * 4 +
                 [_r(RN, H), _b((H, H)), _b((1, H)), _b((H, H))],
        out_specs=[_r(RN, 16), _r(RN, W), _r(RN, W)],
        out_shape=[_sds((N, 16)), _sds((N, W)), _sds((N, W))],
    )(pp, *qs, h1, w1a, b1, w1b)


def _final(pp, qs, h2, ew, eb):
    return pl.pallas_call(
        _final_body,
        grid=(_NGRID,),
        in_specs=[_r(RN, 16)] + [_r(RN, W)] * 4 +
                 [_r(RN, H), _b((H, 8)), _b((1, 8))],
        out_specs=[_r(RN, 16), _r(RN, 8)],
        out_shape=[_sds((N, 16)), _sds((N, 8))],
    )(pp, *qs, h2, ew, eb)


def kernel(h, pos, edge_index, edge_attr, params):
    p = params
    row = edge_index[0]
    col = edge_index[1]

    g0 = p['block_0']['gcl_0']
    e0 = p['block_0']['equiv']
    g1 = p['block_1']['gcl_0']
    e1 = p['block_1']['equiv']

    def rsh(v):
        return v.reshape(1, -1)

    w1a0, w1b0, w1c0 = g0['e_w1'][:H], g0['e_w1'][H:2 * H], g0['e_w1'][2 * H:]
    w1a1, w1b1, w1c1 = g1['e_w1'][:H], g1['e_w1'][H:2 * H], g1['e_w1'][2 * H:]
    wr0 = rsh(w1c0[0] + p['eemb_w'][0] @ w1c0[1:])
    wx0 = p['eemb_w'][1:3] @ w1c0[1:]
    bx0 = rsh(p['eemb_b'] @ w1c0[1:])
    wr1 = rsh(w1c1[0])
    wx1 = w1c1.at[0].set(0.0)
    wout = jnp.concatenate(
        [jnp.zeros((1, 3), _f32), p['eemb_out_w']], axis=0)
    wout = jnp.concatenate([wout, jnp.zeros((H, 5), _f32)], axis=1)
    bout = jnp.concatenate(
        [p['eemb_out_b'], jnp.zeros((5,), _f32)]).reshape(1, 8)
    cw1a0, cw1b0, cw1c0 = e0['c_w1'][:H], e0['c_w1'][H:2 * H], e0['c_w1'][2 * H:]
    cw1a1, cw1b1, cw1c1 = e1['c_w1'][:H], e1['c_w1'][H:2 * H], e1['c_w1'][2 * H:]
    w3_0 = jnp.concatenate([e0['c_w3'], jnp.zeros((H, 7), _f32)], axis=1)
    w3_1 = jnp.concatenate([e1['c_w3'], jnp.zeros((H, 7), _f32)], axis=1)

    z_w = jnp.zeros((RTAIL, W), _f32)
    pp0 = jnp.concatenate([pos, jnp.zeros((N, 13), _f32)], axis=1)

    h64, t1, t2 = _prep0(h, pos, p['emb_w'], rsh(p['emb_b']),
                         w1a0, rsh(g0['e_b1']), w1b0)

    rows = (row[:EH], row[EH:])
    cols = (col[:EH], col[EH:])
    eas = (edge_attr[:EH], edge_attr[EH:])

    gg = [_gather(rows[i], cols[i], t1, t2) for i in (0, 1)]
    m0 = [_edge1_l0(gg[i][0], gg[i][1], eas[i], wr0, wx0, bx0,
                    g0['e_w2'], rsh(g0['e_b2'])) for i in (0, 1)]
    ss = [_scatter(rows[i], m0[i], z_w) for i in (0, 1)]
    h1, ab = _node(h64, [ss[0][0], ss[0][1], ss[1][0], ss[1][1]],
                   g0['n_w1'][:H], g0['n_w1'][H:],
                   rsh(g0['n_b1']), g0['n_w2'], rsh(g0['n_b2']),
                   cw1a0, cw1b0, rsh(e0['c_b1']))
    gg = [_gather(rows[i], cols[i], ab, ab) for i in (0, 1)]
    tr = [_edge2(gg[i][0], gg[i][1], m0[i], cw1c0, e0['c_w2'],
                 rsh(e0['c_b2']), w3_0) for i in (0, 1)]
    qq = [_scatter(rows[i], tr[i], z_w) for i in (0, 1)]
    pp1, t1, t2 = _posprep(pp0, [qq[0][0], qq[0][1], qq[1][0], qq[1][1]],
                           h1, w1a1, rsh(g1['e_b1']), w1b1)

    gg = [_gather(rows[i], cols[i], t1, t2) for i in (0, 1)]
    m1 = [_edge1_l1(gg[i][0], gg[i][1], m0[i], wr1, wx1, g1['e_w2'],
                    rsh(g1['e_b2']), wout, bout) for i in (0, 1)]
    ss = [_scatter(rows[i], m1[i], z_w) for i in (0, 1)]
    h2, ab = _node(h1, [ss[0][0], ss[0][1], ss[1][0], ss[1][1]],
                   g1['n_w1'][:H], g1['n_w1'][H:],
                   rsh(g1['n_b1']), g1['n_w2'], rsh(g1['n_b2']),
                   cw1a1, cw1b1, rsh(e1['c_b1']))
    gg = [_gather(rows[i], cols[i], ab, ab) for i in (0, 1)]
    tr = [_edge2(gg[i][0], gg[i][1], m1[i], cw1c1, e1['c_w2'],
                 rsh(e1['c_b2']), w3_1) for i in (0, 1)]
    qq = [_scatter(rows[i], tr[i], z_w) for i in (0, 1)]
    pos_pad, h_out = _final(pp1, [qq[0][0], qq[0][1], qq[1][0], qq[1][1]],
                            h2, p['emb_out_w'], rsh(p['emb_out_b']))

    ea_pre = jnp.concatenate(
        [m1[0][:, H + 3:H + 6], m1[1][:, H + 3:H + 6]], axis=0)
    return h_out, pos_pad[:, :3], ea_pre


# speedup vs baseline: 3.7715x; 1.0281x over previous
import jax
import jax.numpy as jnp
from jax import lax
from jax.experimental import pallas as pl
from jax.experimental.pallas import tpu as pltpu
from jax.experimental.pallas import tpu_sc as plsc

N = 10000
E = 320000
H = 64
W = 128

_info = plsc.get_sparse_core_info()
NC = _info.num_cores
NS = _info.num_subcores
NW = NC * NS
EH = E // 2
EW = EH // NW
C = 40
K = 5
CH = C * K
ITERS = EW // CH
RPS = 624
RTAIL = N - (NS - 1) * RPS
EPC = EH // NC

_f32 = jnp.float32
_bf16 = jnp.bfloat16


def _silu(x):
    return x / (1.0 + jnp.exp(-x))


def _sds(shape, dtype=jnp.float32):
    return jax.ShapeDtypeStruct(shape, dtype)


_MESH = plsc.VectorSubcoreMesh(core_axis_name="c", subcore_axis_name="s")


def _gather_body(row_h, col_h, t1_h, t2_h, g1_h, g2_h,
                 idx_r, idx_c, bufs0, bufs1, sg0, sg1, sw0, sw1):
    wid = lax.axis_index("s") * NC + lax.axis_index("c")
    tb = wid * EW
    bufs = (bufs0, bufs1)
    sgs = (sg0, sg1)
    sws = (sw0, sw1)

    def g_(st, ch, j):
        src = t1_h.at[idx_r.at[pl.ds(ch * CH + j * C, C)]] if j < K else (
            t2_h.at[idx_c.at[pl.ds(ch * CH + (j - K) * C, C)]])
        return pltpu.make_async_copy(src, bufs[st].at[j], sgs[st])

    def w_(st, ch, j):
        dst = g1_h if j < K else g2_h
        jj = j if j < K else j - K
        return pltpu.make_async_copy(
            bufs[st].at[j], dst.at[pl.ds(tb + ch * CH + jj * C, C)], sws[st])

    def fire_g(st, ch):
        for j in range(2 * K):
            g_(st, ch, j).start()

    def drain_g(st):
        for j in range(2 * K):
            g_(st, 0, j).wait()

    def fire_w(st, ch):
        for j in range(2 * K):
            w_(st, ch, j).start()

    def drain_w(st):
        for j in range(2 * K):
            w_(st, 0, j).wait()

    pltpu.sync_copy(row_h.at[pl.ds(tb, EW)], idx_r)
    pltpu.sync_copy(col_h.at[pl.ds(tb, EW)], idx_c)
    fire_g(0, 0)

    def body(g, carry):
        c0 = 2 * g
        drain_g(0)
        fire_w(0, c0)

        @pl.when(g > 0)
        def _():
            drain_w(1)

        fire_g(1, c0 + 1)
        drain_g(1)
        fire_w(1, c0 + 1)
        drain_w(0)
        fire_g(0, c0 + 2)
        return carry

    lax.fori_loop(0, (ITERS - 1) // 2, body, 0)
    drain_g(0)
    drain_w(1)
    fire_w(0, ITERS - 1)
    drain_w(0)


_gather = pl.kernel(
    _gather_body,
    out_type=[_sds((EH, W)), _sds((EH, W))],
    mesh=_MESH,
    scratch_types=[
        pltpu.VMEM((EW,), jnp.int32), pltpu.VMEM((EW,), jnp.int32),
        pltpu.VMEM((2 * K, C, W), _f32), pltpu.VMEM((2 * K, C, W), _f32),
        pltpu.SemaphoreType.DMA, pltpu.SemaphoreType.DMA,
        pltpu.SemaphoreType.DMA, pltpu.SemaphoreType.DMA,
    ],
)


def _scatter_body(row_h, data_h, zeros_h, out0_h, out1_h,
                  idx_a, idx_b, dat_a, dat_b, shared, sa, sb):
    c = lax.axis_index("c")
    s = lax.axis_index("s")
    r0 = s * RPS
    last = NS - 1
    tb = c * EPC + s * EW

    def fire(buf_idx, buf_dat, sem, base):
        pltpu.make_async_copy(
            row_h.at[pl.ds(base, C)], buf_idx, sem).start()
        pltpu.make_async_copy(
            data_h.at[pl.ds(base, C)], buf_dat, sem).start()

    def drain(buf_idx, buf_dat, sem):
        pltpu.make_async_copy(
            row_h.at[pl.ds(tb, C)], buf_idx, sem).wait()
        pltpu.make_async_copy(
            data_h.at[pl.ds(tb, C)], buf_dat, sem).wait()

    def adds(buf_idx, buf_dat):
        pltpu.sync_copy(buf_dat, shared.at[buf_idx], add=True)

    @pl.when(s != last)
    def _():
        pltpu.sync_copy(zeros_h.at[pl.ds(0, RPS)], shared.at[pl.ds(r0, RPS)])

    @pl.when(s == last)
    def _():
        pltpu.sync_copy(zeros_h, shared.at[pl.ds(last * RPS, RTAIL)])

    plsc.subcore_barrier()

    fire(idx_a, dat_a, sa, tb)

    def body(g, carry):
        b0 = tb + 2 * g * C
        drain(idx_a, dat_a, sa)
        fire(idx_b, dat_b, sb, b0 + C)
        adds(idx_a, dat_a)
        drain(idx_b, dat_b, sb)
        fire(idx_a, dat_a, sa, b0 + 2 * C)
        adds(idx_b, dat_b)
        return carry

    lax.fori_loop(0, (EW // C - 1) // 2, body, 0)
    drain(idx_a, dat_a, sa)
    adds(idx_a, dat_a)
    plsc.subcore_barrier()

    @pl.when((c == 0) & (s != last))
    def _():
        pltpu.sync_copy(shared.at[pl.ds(r0, RPS)], out0_h.at[pl.ds(r0, RPS)])

    @pl.when((c == 0) & (s == last))
    def _():
        pltpu.sync_copy(shared.at[pl.ds(last * RPS, RTAIL)],
                        out0_h.at[pl.ds(last * RPS, RTAIL)])

    @pl.when((c == 1) & (s != last))
    def _():
        pltpu.sync_copy(shared.at[pl.ds(r0, RPS)], out1_h.at[pl.ds(r0, RPS)])

    @pl.when((c == 1) & (s == last))
    def _():
        pltpu.sync_copy(shared.at[pl.ds(last * RPS, RTAIL)],
                        out1_h.at[pl.ds(last * RPS, RTAIL)])


_scatter = pl.kernel(
    _scatter_body,
    out_type=[_sds((N, W)), _sds((N, W))],
    mesh=_MESH,
    scratch_types=[
        pltpu.VMEM((C,), jnp.int32), pltpu.VMEM((C,), jnp.int32),
        pltpu.VMEM((C, W), _f32), pltpu.VMEM((C, W), _f32),
        pltpu.VMEM_SHARED((N, W), _f32),
        pltpu.SemaphoreType.DMA, pltpu.SemaphoreType.DMA,
    ],
)


BE = 3200
RN = 2000
_EGRID = EH // BE
_NGRID = N // RN


def _b(shape):
    return pl.BlockSpec(shape, lambda i: tuple(0 for _ in shape))


def _r(rows, cols):
    return pl.BlockSpec((rows, cols), lambda i: (i, 0))


def _dot(a, b):
    return jnp.dot(a, b, preferred_element_type=_f32)


def _bdot(a, b):
    return jnp.dot(a.astype(_bf16), b.astype(_bf16),
                   preferred_element_type=_f32)


def _zeros(rows, cols):
    return jnp.zeros((rows, cols), _f32)


def _prep0_body(h_r, pos_r, embw_r, embb_r, w1a_r, b1_r, w1b_r,
                h64_o, t1_o, t2_o):
    h64 = _dot(h_r[...], embw_r[...]) + embb_r[...]
    h64_o[...] = h64
    pospad = jnp.concatenate([pos_r[...], _zeros(RN, 13)], axis=1)
    a = _dot(h64, w1a_r[...]) + b1_r[...]
    b = _dot(h64, w1b_r[...])
    t1_o[...] = jnp.concatenate([a, pospad, _zeros(RN, 48)], axis=1)
    t2_o[...] = jnp.concatenate([b, pospad, _zeros(RN, 48)], axis=1)


def _edge_geom(g1, g2):
    cd3 = g1[:, H:H + 3] - g2[:, H:H + 3]
    radial = jnp.sum(cd3 * cd3, axis=1, keepdims=True)
    cdiff = cd3 / (jnp.sqrt(radial + 1e-8) + 1.0)
    return radial, cdiff


def _edge1_body_l0(g1_r, g2_r, x_r, wr_r, wx_r, bx_r, w2_r, b2_r, m_o):
    g1 = g1_r[...]
    g2 = g2_r[...]
    radial, cdiff = _edge_geom(g1, g2)
    pre1 = (g1[:, :H] + g2[:, :H] + radial * wr_r[...]
            + _bdot(x_r[...], wx_r[...]) + bx_r[...])
    mij = _silu(_bdot(_silu(pre1), w2_r[...]) + b2_r[...])
    m_o[...] = jnp.concatenate([mij, cdiff, _zeros(BE, W - H - 3)], axis=1)


def _edge1_body_l1(g1_r, g2_r, x_r, wr_r, wx_r, w2_r, b2_r, wout_r, bout_r,
                   m_o):
    g1 = g1_r[...]
    g2 = g2_r[...]
    radial, cdiff = _edge_geom(g1, g2)
    pre1 = (g1[:, :H] + g2[:, :H] + radial * wr_r[...]
            + _bdot(x_r[:, :H], wx_r[...]))
    mij = _silu(_bdot(_silu(pre1), w2_r[...]) + b2_r[...])
    eaout = _bdot(mij, wout_r[...]) + bout_r[...]
    m_o[...] = jnp.concatenate(
        [mij, cdiff, eaout, _zeros(BE, W - H - 3 - 8)], axis=1)


def _edge2_body(g1_r, g2_r, m_r, wm_r, w2_r, b2_r, w3_r, t_o):
    m = m_r[...]
    pre = g1_r[:, :H] + g2_r[:, H:] + _bdot(m[:, :H], wm_r[...])
    u = _silu(_bdot(_silu(pre), w2_r[...]) + b2_r[...])
    phi = _bdot(u, w3_r[...])[:, 0:1]
    trans = m[:, H:H + 3] * phi
    t_o[...] = jnp.concatenate([trans, _zeros(BE, W - 3)], axis=1)


def _node_body(h_r, p0_r, p1_r, p2_r, p3_r, w1a_r, w1b_r, b1_r, w2_r, b2_r,
               cw1a_r, cw1b_r, cb1_r, hn_o, ab_o):
    agg = (p0_r[:, :H] + p1_r[:, :H] + p2_r[:, :H] + p3_r[:, :H]) * 0.01
    pre = _dot(h_r[...], w1a_r[...]) + _dot(agg, w1b_r[...]) + b1_r[...]
    hn = h_r[...] + _dot(_silu(pre), w2_r[...]) + b2_r[...]
    hn_o[...] = hn
    aeq = _dot(hn, cw1a_r[...]) + cb1_r[...]
    beq = _dot(hn, cw1b_r[...])
    ab_o[...] = jnp.concatenate([aeq, beq], axis=1)


def _posprep_body(pp_r, q0_r, q1_r, q2_r, q3_r, h_r, w1a_r, b1_r, w1b_r,
                  pp_o, t1_o, t2_o):
    pp1 = pp_r[...] + (q0_r[:, :16] + q1_r[:, :16]
                       + q2_r[:, :16] + q3_r[:, :16]) * 0.01
    pp_o[...] = pp1
    a = _dot(h_r[...], w1a_r[...]) + b1_r[...]
    b = _dot(h_r[...], w1b_r[...])
    t1_o[...] = jnp.concatenate([a, pp1, _zeros(RN, 48)], axis=1)
    t2_o[...] = jnp.concatenate([b, pp1, _zeros(RN, 48)], axis=1)


def _final_body(pp_r, q0_r, q1_r, q2_r, q3_r, h_r, ew_r, eb_r,
                pos_o, hout_o):
    pos_o[...] = pp_r[...] + (q0_r[:, :16] + q1_r[:, :16]
                              + q2_r[:, :16] + q3_r[:, :16]) * 0.01
    hout_o[...] = _dot(h_r[...], ew_r[...]) + eb_r[...]


def _prep0(h, pos, embw, embb, w1a, b1, w1b):
    return pl.pallas_call(
        _prep0_body,
        grid=(_NGRID,),
        in_specs=[_r(RN, 8), _r(RN, 3), _b((8, H)), _b((1, H)),
                  _b((H, H)), _b((1, H)), _b((H, H))],
        out_specs=[_r(RN, H), _r(RN, W), _r(RN, W)],
        out_shape=[_sds((N, H)), _sds((N, W)), _sds((N, W))],
    )(h, pos, embw, embb, w1a, b1, w1b)


def _edge1_l0(g1, g2, x, wr, wx, bx, w2, b2):
    return pl.pallas_call(
        _edge1_body_l0,
        grid=(_EGRID,),
        in_specs=[_r(BE, W), _r(BE, W), _r(BE, 2),
                  _b((1, H)), _b((2, H)), _b((1, H)), _b((H, H)), _b((1, H))],
        out_specs=[_r(BE, W)],
        out_shape=[_sds((EH, W))],
    )(g1, g2, x, wr, wx, bx, w2, b2)[0]


def _edge1_l1(g1, g2, x, wr, wx, w2, b2, wout, bout):
    return pl.pallas_call(
        _edge1_body_l1,
        grid=(_EGRID,),
        in_specs=[_r(BE, W), _r(BE, W), _r(BE, W),
                  _b((1, H)), _b((H, H)), _b((H, H)), _b((1, H)),
                  _b((H, 8)), _b((1, 8))],
        out_specs=[_r(BE, W)],
        out_shape=[_sds((EH, W))],
    )(g1, g2, x, wr, wx, w2, b2, wout, bout)[0]


def _edge2(g1, g2, m, wm, w2, b2, w3):
    return pl.pallas_call(
        _edge2_body,
        grid=(_EGRID,),
        in_specs=[_r(BE, W), _r(BE, W), _r(BE, W),
                  _b((H, H)), _b((H, H)), _b((1, H)), _b((H, 8))],
        out_specs=[_r(BE, W)],
        out_shape=[_sds((EH, W))],
    )(g1, g2, m, wm, w2, b2, w3)[0]


def _node(h64, ps, w1a, w1b, b1, w2, b2, cw1a, cw1b, cb1):
    return pl.pallas_call(
        _node_body,
        grid=(_NGRID,),
        in_specs=[_r(RN, H)] + [_r(RN, W)] * 4 +
                 [_b((H, H)), _b((H, H)), _b((1, H)), _b((H, H)), _b((1, H)),
                  _b((H, H)), _b((H, H)), _b((1, H))],
        out_specs=[_r(RN, H), _r(RN, W)],
        out_shape=[_sds((N, H)), _sds((N, W))],
    )(h64, *ps, w1a, w1b, b1, w2, b2, cw1a, cw1b, cb1)


def _posprep(pp, qs, h1, w1a, b1, w1b):
    return pl.pallas_call(
        _posprep_body,
        grid=(_NGRID,),
        in_specs=[_r(RN, 16)] + [_r(RN, W)] * 4 +
                 [_r(RN, H), _b((H, H)), _b((1, H)), _b((H, H))],
        out_specs=[_r(RN, 16), _r(RN, W), _r(RN, W)],
        out_shape=[_sds((N, 16)), _sds((N, W)), _sds((N, W))],
    )(pp, *qs, h1, w1a, b1, w1b)


def _final(pp, qs, h2, ew, eb):
    return pl.pallas_call(
        _final_body,
        grid=(_NGRID,),
        in_specs=[_r(RN, 16)] + [_r(RN, W)] * 4 +
                 [_r(RN, H), _b((H, 8)), _b((1, 8))],
        out_specs=[_r(RN, 16), _r(RN, 8)],
        out_shape=[_sds((N, 16)), _sds((N, 8))],
    )(pp, *qs, h2, ew, eb)


def kernel(h, pos, edge_index, edge_attr, params):
    p = params
    row = edge_index[0]
    col = edge_index[1]

    g0 = p['block_0']['gcl_0']
    e0 = p['block_0']['equiv']
    g1 = p['block_1']['gcl_0']
    e1 = p['block_1']['equiv']

    def rsh(v):
        return v.reshape(1, -1)

    w1a0, w1b0, w1c0 = g0['e_w1'][:H], g0['e_w1'][H:2 * H], g0['e_w1'][2 * H:]
    w1a1, w1b1, w1c1 = g1['e_w1'][:H], g1['e_w1'][H:2 * H], g1['e_w1'][2 * H:]
    wr0 = rsh(w1c0[0] + p['eemb_w'][0] @ w1c0[1:])
    wx0 = p['eemb_w'][1:3] @ w1c0[1:]
    bx0 = rsh(p['eemb_b'] @ w1c0[1:])
    wr1 = rsh(w1c1[0])
    wx1 = w1c1.at[0].set(0.0)
    wout = jnp.concatenate(
        [jnp.zeros((1, 3), _f32), p['eemb_out_w']], axis=0)
    wout = jnp.concatenate([wout, jnp.zeros((H, 5), _f32)], axis=1)
    bout = jnp.concatenate(
        [p['eemb_out_b'], jnp.zeros((5,), _f32)]).reshape(1, 8)
    cw1a0, cw1b0, cw1c0 = e0['c_w1'][:H], e0['c_w1'][H:2 * H], e0['c_w1'][2 * H:]
    cw1a1, cw1b1, cw1c1 = e1['c_w1'][:H], e1['c_w1'][H:2 * H], e1['c_w1'][2 * H:]
    w3_0 = jnp.concatenate([e0['c_w3'], jnp.zeros((H, 7), _f32)], axis=1)
    w3_1 = jnp.concatenate([e1['c_w3'], jnp.zeros((H, 7), _f32)], axis=1)

    z_w = jnp.zeros((RTAIL, W), _f32)
    pp0 = jnp.concatenate([pos, jnp.zeros((N, 13), _f32)], axis=1)

    h64, t1, t2 = _prep0(h, pos, p['emb_w'], rsh(p['emb_b']),
                         w1a0, rsh(g0['e_b1']), w1b0)

    rows = (row[:EH], row[EH:])
    cols = (col[:EH], col[EH:])
    eas = (edge_attr[:EH], edge_attr[EH:])

    gg = [_gather(rows[i], cols[i], t1, t2) for i in (0, 1)]
    m0 = [_edge1_l0(gg[i][0], gg[i][1], eas[i], wr0, wx0, bx0,
                    g0['e_w2'], rsh(g0['e_b2'])) for i in (0, 1)]
    ss = [_scatter(rows[i], m0[i], z_w) for i in (0, 1)]
    h1, ab = _node(h64, [ss[0][0], ss[0][1], ss[1][0], ss[1][1]],
                   g0['n_w1'][:H], g0['n_w1'][H:],
                   rsh(g0['n_b1']), g0['n_w2'], rsh(g0['n_b2']),
                   cw1a0, cw1b0, rsh(e0['c_b1']))
    gg = [_gather(rows[i], cols[i], ab, ab) for i in (0, 1)]
    tr = [_edge2(gg[i][0], gg[i][1], m0[i], cw1c0, e0['c_w2'],
                 rsh(e0['c_b2']), w3_0) for i in (0, 1)]
    qq = [_scatter(rows[i], tr[i], z_w) for i in (0, 1)]
    pp1, t1, t2 = _posprep(pp0, [qq[0][0], qq[0][1], qq[1][0], qq[1][1]],
                           h1, w1a1, rsh(g1['e_b1']), w1b1)

    gg = [_gather(rows[i], cols[i], t1, t2) for i in (0, 1)]
    m1 = [_edge1_l1(gg[i][0], gg[i][1], m0[i], wr1, wx1, g1['e_w2'],
                    rsh(g1['e_b2']), wout, bout) for i in (0, 1)]
    ss = [_scatter(rows[i], m1[i], z_w) for i in (0, 1)]
    h2, ab = _node(h1, [ss[0][0], ss[0][1], ss[1][0], ss[1][1]],
                   g1['n_w1'][:H], g1['n_w1'][H:],
                   rsh(g1['n_b1']), g1['n_w2'], rsh(g1['n_b2']),
                   cw1a1, cw1b1, rsh(e1['c_b1']))
    gg = [_gather(rows[i], cols[i], ab, ab) for i in (0, 1)]
    tr = [_edge2(gg[i][0], gg[i][1], m1[i], cw1c1, e1['c_w2'],
                 rsh(e1['c_b2']), w3_1) for i in (0, 1)]
    qq = [_scatter(rows[i], tr[i], z_w) for i in (0, 1)]
    pos_pad, h_out = _final(pp1, [qq[0][0], qq[0][1], qq[1][0], qq[1][1]],
                            h2, p['emb_out_w'], rsh(p['emb_out_b']))

    ea_pre = jnp.concatenate(
        [m1[0][:, H + 3:H + 6], m1[1][:, H + 3:H + 6]], axis=0)
    return h_out, pos_pad[:, :3], ea_pre


# speedup vs baseline: 3.9332x; 1.0429x over previous
import jax
import jax.numpy as jnp
from jax import lax
from jax.experimental import pallas as pl
from jax.experimental.pallas import tpu as pltpu
from jax.experimental.pallas import tpu_sc as plsc

N = 10000
E = 320000
H = 64
W = 128

_info = plsc.get_sparse_core_info()
NC = _info.num_cores
NS = _info.num_subcores
NW = NC * NS
EH = E // 2
EW = EH // NW
C = 40
K = 5
CH = C * K
ITERS = EW // CH
RPS = 624
RTAIL = N - (NS - 1) * RPS
EPC = EH // NC

_f32 = jnp.float32
_bf16 = jnp.bfloat16


def _silu(x):
    return x / (1.0 + jnp.exp(-x))


def _sds(shape, dtype=jnp.float32):
    return jax.ShapeDtypeStruct(shape, dtype)


_MESH = plsc.VectorSubcoreMesh(core_axis_name="c", subcore_axis_name="s")


def _gather_body(row_h, col_h, t1_h, t2_h, g1_h, g2_h,
                 idx_r, idx_c, bufs0, bufs1, sg0, sg1, sw0, sw1):
    wid = lax.axis_index("s") * NC + lax.axis_index("c")
    tb = wid * EW
    bufs = (bufs0, bufs1)
    sgs = (sg0, sg1)
    sws = (sw0, sw1)

    def g_(st, ch, j):
        src = t1_h.at[idx_r.at[pl.ds(ch * CH + j * C, C)]] if j < K else (
            t2_h.at[idx_c.at[pl.ds(ch * CH + (j - K) * C, C)]])
        return pltpu.make_async_copy(src, bufs[st].at[j], sgs[st])

    def w_(st, ch, j):
        dst = g1_h if j < K else g2_h
        jj = j if j < K else j - K
        return pltpu.make_async_copy(
            bufs[st].at[j], dst.at[pl.ds(tb + ch * CH + jj * C, C)], sws[st])

    def fire_g(st, ch):
        for j in range(2 * K):
            g_(st, ch, j).start()

    def drain_g(st):
        for j in range(2 * K):
            g_(st, 0, j).wait()

    def fire_w(st, ch):
        for j in range(2 * K):
            w_(st, ch, j).start()

    def drain_w(st):
        for j in range(2 * K):
            w_(st, 0, j).wait()

    pltpu.sync_copy(row_h.at[pl.ds(tb, EW)], idx_r)
    pltpu.sync_copy(col_h.at[pl.ds(tb, EW)], idx_c)
    fire_g(0, 0)

    def body(g, carry):
        c0 = 2 * g
        drain_g(0)
        fire_w(0, c0)

        @pl.when(g > 0)
        def _():
            drain_w(1)

        fire_g(1, c0 + 1)
        drain_g(1)
        fire_w(1, c0 + 1)
        drain_w(0)
        fire_g(0, c0 + 2)
        return carry

    lax.fori_loop(0, (ITERS - 1) // 2, body, 0)
    drain_g(0)
    drain_w(1)
    fire_w(0, ITERS - 1)
    drain_w(0)


_gather = pl.kernel(
    _gather_body,
    out_type=[_sds((EH, W)), _sds((EH, W))],
    mesh=_MESH,
    scratch_types=[
        pltpu.VMEM((EW,), jnp.int32), pltpu.VMEM((EW,), jnp.int32),
        pltpu.VMEM((2 * K, C, W), _f32), pltpu.VMEM((2 * K, C, W), _f32),
        pltpu.SemaphoreType.DMA, pltpu.SemaphoreType.DMA,
        pltpu.SemaphoreType.DMA, pltpu.SemaphoreType.DMA,
    ],
)


def _scatter_body(row_h, data_h, zeros_h, out0_h, out1_h,
                  idx_a, idx_b, dat_a, dat_b, shared, sa, sb, saa, sab):
    c = lax.axis_index("c")
    s = lax.axis_index("s")
    r0 = s * RPS
    last = NS - 1
    tb = c * EPC + s * EW

    def fire(buf_idx, buf_dat, sem, base):
        pltpu.make_async_copy(
            row_h.at[pl.ds(base, C)], buf_idx, sem).start()
        pltpu.make_async_copy(
            data_h.at[pl.ds(base, C)], buf_dat, sem).start()

    def drain(buf_idx, buf_dat, sem):
        pltpu.make_async_copy(
            row_h.at[pl.ds(tb, C)], buf_idx, sem).wait()
        pltpu.make_async_copy(
            data_h.at[pl.ds(tb, C)], buf_dat, sem).wait()

    def add_cp(buf_idx, buf_dat, sem):
        return pltpu.async_copy(buf_dat, shared.at[buf_idx], sem, add=True)

    @pl.when(s != last)
    def _():
        pltpu.sync_copy(zeros_h.at[pl.ds(0, RPS)], shared.at[pl.ds(r0, RPS)])

    @pl.when(s == last)
    def _():
        pltpu.sync_copy(zeros_h, shared.at[pl.ds(last * RPS, RTAIL)])

    plsc.subcore_barrier()

    nch = EW // C
    fire(idx_a, dat_a, sa, tb)
    fire(idx_b, dat_b, sb, tb + C)

    def body(g, carry):
        b0 = tb + 2 * g * C
        drain(idx_a, dat_a, sa)
        ha = add_cp(idx_a, dat_a, saa)
        drain(idx_b, dat_b, sb)
        hb = add_cp(idx_b, dat_b, sab)
        ha.wait()
        fire(idx_a, dat_a, sa, b0 + 2 * C)
        hb.wait()

        @pl.when(2 * g + 3 <= nch - 1)
        def _():
            fire(idx_b, dat_b, sb, b0 + 3 * C)

        return carry

    lax.fori_loop(0, (nch - 1) // 2, body, 0)
    drain(idx_a, dat_a, sa)
    add_cp(idx_a, dat_a, saa).wait()
    plsc.subcore_barrier()

    @pl.when((c == 0) & (s != last))
    def _():
        pltpu.sync_copy(shared.at[pl.ds(r0, RPS)], out0_h.at[pl.ds(r0, RPS)])

    @pl.when((c == 0) & (s == last))
    def _():
        pltpu.sync_copy(shared.at[pl.ds(last * RPS, RTAIL)],
                        out0_h.at[pl.ds(last * RPS, RTAIL)])

    @pl.when((c == 1) & (s != last))
    def _():
        pltpu.sync_copy(shared.at[pl.ds(r0, RPS)], out1_h.at[pl.ds(r0, RPS)])

    @pl.when((c == 1) & (s == last))
    def _():
        pltpu.sync_copy(shared.at[pl.ds(last * RPS, RTAIL)],
                        out1_h.at[pl.ds(last * RPS, RTAIL)])


_scatter = pl.kernel(
    _scatter_body,
    out_type=[_sds((N, W)), _sds((N, W))],
    mesh=_MESH,
    scratch_types=[
        pltpu.VMEM((C,), jnp.int32), pltpu.VMEM((C,), jnp.int32),
        pltpu.VMEM((C, W), _f32), pltpu.VMEM((C, W), _f32),
        pltpu.VMEM_SHARED((N, W), _f32),
        pltpu.SemaphoreType.DMA, pltpu.SemaphoreType.DMA,
        pltpu.SemaphoreType.DMA, pltpu.SemaphoreType.DMA,
    ],
)


BE = 3200
RN = 2000
_EGRID = EH // BE
_NGRID = N // RN


def _b(shape):
    return pl.BlockSpec(shape, lambda i: tuple(0 for _ in shape))


def _r(rows, cols):
    return pl.BlockSpec((rows, cols), lambda i: (i, 0))


def _dot(a, b):
    return jnp.dot(a, b, preferred_element_type=_f32)


def _bdot(a, b):
    return jnp.dot(a.astype(_bf16), b.astype(_bf16),
                   preferred_element_type=_f32)


def _zeros(rows, cols):
    return jnp.zeros((rows, cols), _f32)


def _prep0_body(h_r, pos_r, embw_r, embb_r, w1a_r, b1_r, w1b_r,
                h64_o, t1_o, t2_o):
    h64 = _dot(h_r[...], embw_r[...]) + embb_r[...]
    h64_o[...] = h64
    pospad = jnp.concatenate([pos_r[...], _zeros(RN, 13)], axis=1)
    a = _dot(h64, w1a_r[...]) + b1_r[...]
    b = _dot(h64, w1b_r[...])
    t1_o[...] = jnp.concatenate([a, pospad, _zeros(RN, 48)], axis=1)
    t2_o[...] = jnp.concatenate([b, pospad, _zeros(RN, 48)], axis=1)


def _edge_geom(g1, g2):
    cd3 = g1[:, H:H + 3] - g2[:, H:H + 3]
    radial = jnp.sum(cd3 * cd3, axis=1, keepdims=True)
    cdiff = cd3 / (jnp.sqrt(radial + 1e-8) + 1.0)
    return radial, cdiff


def _edge1_body_l0(g1_r, g2_r, x_r, wr_r, wx_r, bx_r, w2_r, b2_r, m_o):
    g1 = g1_r[...]
    g2 = g2_r[...]
    radial, cdiff = _edge_geom(g1, g2)
    pre1 = (g1[:, :H] + g2[:, :H] + radial * wr_r[...]
            + _bdot(x_r[...], wx_r[...]) + bx_r[...])
    mij = _silu(_bdot(_silu(pre1), w2_r[...]) + b2_r[...])
    m_o[...] = jnp.concatenate([mij, cdiff, _zeros(BE, W - H - 3)], axis=1)


def _edge1_body_l1(g1_r, g2_r, x_r, wr_r, wx_r, w2_r, b2_r, wout_r, bout_r,
                   m_o):
    g1 = g1_r[...]
    g2 = g2_r[...]
    radial, cdiff = _edge_geom(g1, g2)
    pre1 = (g1[:, :H] + g2[:, :H] + radial * wr_r[...]
            + _bdot(x_r[:, :H], wx_r[...]))
    mij = _silu(_bdot(_silu(pre1), w2_r[...]) + b2_r[...])
    eaout = _bdot(mij, wout_r[...]) + bout_r[...]
    m_o[...] = jnp.concatenate(
        [mij, cdiff, eaout, _zeros(BE, W - H - 3 - 8)], axis=1)


def _edge2_body(g1_r, g2_r, m_r, wm_r, w2_r, b2_r, w3_r, t_o):
    m = m_r[...]
    pre = g1_r[:, :H] + g2_r[:, H:] + _bdot(m[:, :H], wm_r[...])
    u = _silu(_bdot(_silu(pre), w2_r[...]) + b2_r[...])
    phi = _bdot(u, w3_r[...])[:, 0:1]
    trans = m[:, H:H + 3] * phi
    t_o[...] = jnp.concatenate([trans, _zeros(BE, W - 3)], axis=1)


def _node_body(h_r, p0_r, p1_r, p2_r, p3_r, w1a_r, w1b_r, b1_r, w2_r, b2_r,
               cw1a_r, cw1b_r, cb1_r, hn_o, ab_o):
    agg = (p0_r[:, :H] + p1_r[:, :H] + p2_r[:, :H] + p3_r[:, :H]) * 0.01
    pre = _dot(h_r[...], w1a_r[...]) + _dot(agg, w1b_r[...]) + b1_r[...]
    hn = h_r[...] + _dot(_silu(pre), w2_r[...]) + b2_r[...]
    hn_o[...] = hn
    aeq = _dot(hn, cw1a_r[...]) + cb1_r[...]
    beq = _dot(hn, cw1b_r[...])
    ab_o[...] = jnp.concatenate([aeq, beq], axis=1)


def _posprep_body(pp_r, q0_r, q1_r, q2_r, q3_r, h_r, w1a_r, b1_r, w1b_r,
                  pp_o, t1_o, t2_o):
    pp1 = pp_r[...] + (q0_r[:, :16] + q1_r[:, :16]
                       + q2_r[:, :16] + q3_r[:, :16]) * 0.01
    pp_o[...] = pp1
    a = _dot(h_r[...], w1a_r[...]) + b1_r[...]
    b = _dot(h_r[...], w1b_r[...])
    t1_o[...] = jnp.concatenate([a, pp1, _zeros(RN, 48)], axis=1)
    t2_o[...] = jnp.concatenate([b, pp1, _zeros(RN, 48)], axis=1)


def _final_body(pp_r, q0_r, q1_r, q2_r, q3_r, h_r, ew_r, eb_r,
                pos_o, hout_o):
    pos_o[...] = pp_r[...] + (q0_r[:, :16] + q1_r[:, :16]
                              + q2_r[:, :16] + q3_r[:, :16]) * 0.01
    hout_o[...] = _dot(h_r[...], ew_r[...]) + eb_r[...]


def _prep0(h, pos, embw, embb, w1a, b1, w1b):
    return pl.pallas_call(
        _prep0_body,
        grid=(_NGRID,),
        in_specs=[_r(RN, 8), _r(RN, 3), _b((8, H)), _b((1, H)),
                  _b((H, H)), _b((1, H)), _b((H, H))],
        out_specs=[_r(RN, H), _r(RN, W), _r(RN, W)],
        out_shape=[_sds((N, H)), _sds((N, W)), _sds((N, W))],
    )(h, pos, embw, embb, w1a, b1, w1b)


def _edge1_l0(g1, g2, x, wr, wx, bx, w2, b2):
    return pl.pallas_call(
        _edge1_body_l0,
        grid=(_EGRID,),
        in_specs=[_r(BE, W), _r(BE, W), _r(BE, 2),
                  _b((1, H)), _b((2, H)), _b((1, H)), _b((H, H)), _b((1, H))],
        out_specs=[_r(BE, W)],
        out_shape=[_sds((EH, W))],
    )(g1, g2, x, wr, wx, bx, w2, b2)[0]


def _edge1_l1(g1, g2, x, wr, wx, w2, b2, wout, bout):
    return pl.pallas_call(
        _edge1_body_l1,
        grid=(_EGRID,),
        in_specs=[_r(BE, W), _r(BE, W), _r(BE, W),
                  _b((1, H)), _b((H, H)), _b((H, H)), _b((1, H)),
                  _b((H, 8)), _b((1, 8))],
        out_specs=[_r(BE, W)],
        out_shape=[_sds((EH, W))],
    )(g1, g2, x, wr, wx, w2, b2, wout, bout)[0]


def _edge2(g1, g2, m, wm, w2, b2, w3):
    return pl.pallas_call(
        _edge2_body,
        grid=(_EGRID,),
        in_specs=[_r(BE, W), _r(BE, W), _r(BE, W),
                  _b((H, H)), _b((H, H)), _b((1, H)), _b((H, 8))],
        out_specs=[_r(BE, W)],
        out_shape=[_sds((EH, W))],
    )(g1, g2, m, wm, w2, b2, w3)[0]


def _node(h64, ps, w1a, w1b, b1, w2, b2, cw1a, cw1b, cb1):
    return pl.pallas_call(
        _node_body,
        grid=(_NGRID,),
        in_specs=[_r(RN, H)] + [_r(RN, W)] * 4 +
                 [_b((H, H)), _b((H, H)), _b((1, H)), _b((H, H)), _b((1, H)),
                  _b((H, H)), _b((H, H)), _b((1, H))],
        out_specs=[_r(RN, H), _r(RN, W)],
        out_shape=[_sds((N, H)), _sds((N, W))],
    )(h64, *ps, w1a, w1b, b1, w2, b2, cw1a, cw1b, cb1)


def _posprep(pp, qs, h1, w1a, b1, w1b):
    return pl.pallas_call(
        _posprep_body,
        grid=(_NGRID,),
        in_specs=[_r(RN, 16)] + [_r(RN, W)] * 4 +
                 [_r(RN, H), _b((H, H)), _b((1, H)), _b((H, H))],
        out_specs=[_r(RN, 16), _r(RN, W), _r(RN, W)],
        out_shape=[_sds((N, 16)), _sds((N, W)), _sds((N, W))],
    )(pp, *qs, h1, w1a, b1, w1b)


def _final(pp, qs, h2, ew, eb):
    return pl.pallas_call(
        _final_body,
        grid=(_NGRID,),
        in_specs=[_r(RN, 16)] + [_r(RN, W)] * 4 +
                 [_r(RN, H), _b((H, 8)), _b((1, 8))],
        out_specs=[_r(RN, 16), _r(RN, 8)],
        out_shape=[_sds((N, 16)), _sds((N, 8))],
    )(pp, *qs, h2, ew, eb)


def kernel(h, pos, edge_index, edge_attr, params):
    p = params
    row = edge_index[0]
    col = edge_index[1]

    g0 = p['block_0']['gcl_0']
    e0 = p['block_0']['equiv']
    g1 = p['block_1']['gcl_0']
    e1 = p['block_1']['equiv']

    def rsh(v):
        return v.reshape(1, -1)

    w1a0, w1b0, w1c0 = g0['e_w1'][:H], g0['e_w1'][H:2 * H], g0['e_w1'][2 * H:]
    w1a1, w1b1, w1c1 = g1['e_w1'][:H], g1['e_w1'][H:2 * H], g1['e_w1'][2 * H:]
    wr0 = rsh(w1c0[0] + p['eemb_w'][0] @ w1c0[1:])
    wx0 = p['eemb_w'][1:3] @ w1c0[1:]
    bx0 = rsh(p['eemb_b'] @ w1c0[1:])
    wr1 = rsh(w1c1[0])
    wx1 = w1c1.at[0].set(0.0)
    wout = jnp.concatenate(
        [jnp.zeros((1, 3), _f32), p['eemb_out_w']], axis=0)
    wout = jnp.concatenate([wout, jnp.zeros((H, 5), _f32)], axis=1)
    bout = jnp.concatenate(
        [p['eemb_out_b'], jnp.zeros((5,), _f32)]).reshape(1, 8)
    cw1a0, cw1b0, cw1c0 = e0['c_w1'][:H], e0['c_w1'][H:2 * H], e0['c_w1'][2 * H:]
    cw1a1, cw1b1, cw1c1 = e1['c_w1'][:H], e1['c_w1'][H:2 * H], e1['c_w1'][2 * H:]
    w3_0 = jnp.concatenate([e0['c_w3'], jnp.zeros((H, 7), _f32)], axis=1)
    w3_1 = jnp.concatenate([e1['c_w3'], jnp.zeros((H, 7), _f32)], axis=1)

    z_w = jnp.zeros((RTAIL, W), _f32)
    pp0 = jnp.concatenate([pos, jnp.zeros((N, 13), _f32)], axis=1)

    h64, t1, t2 = _prep0(h, pos, p['emb_w'], rsh(p['emb_b']),
                         w1a0, rsh(g0['e_b1']), w1b0)

    rows = (row[:EH], row[EH:])
    cols = (col[:EH], col[EH:])
    eas = (edge_attr[:EH], edge_attr[EH:])

    gg = [_gather(rows[i], cols[i], t1, t2) for i in (0, 1)]
    m0 = [_edge1_l0(gg[i][0], gg[i][1], eas[i], wr0, wx0, bx0,
                    g0['e_w2'], rsh(g0['e_b2'])) for i in (0, 1)]
    ss = [_scatter(rows[i], m0[i], z_w) for i in (0, 1)]
    h1, ab = _node(h64, [ss[0][0], ss[0][1], ss[1][0], ss[1][1]],
                   g0['n_w1'][:H], g0['n_w1'][H:],
                   rsh(g0['n_b1']), g0['n_w2'], rsh(g0['n_b2']),
                   cw1a0, cw1b0, rsh(e0['c_b1']))
    gg = [_gather(rows[i], cols[i], ab, ab) for i in (0, 1)]
    tr = [_edge2(gg[i][0], gg[i][1], m0[i], cw1c0, e0['c_w2'],
                 rsh(e0['c_b2']), w3_0) for i in (0, 1)]
    qq = [_scatter(rows[i], tr[i], z_w) for i in (0, 1)]
    pp1, t1, t2 = _posprep(pp0, [qq[0][0], qq[0][1], qq[1][0], qq[1][1]],
                           h1, w1a1, rsh(g1['e_b1']), w1b1)

    gg = [_gather(rows[i], cols[i], t1, t2) for i in (0, 1)]
    m1 = [_edge1_l1(gg[i][0], gg[i][1], m0[i], wr1, wx1, g1['e_w2'],
                    rsh(g1['e_b2']), wout, bout) for i in (0, 1)]
    ss = [_scatter(rows[i], m1[i], z_w) for i in (0, 1)]
    h2, ab = _node(h1, [ss[0][0], ss[0][1], ss[1][0], ss[1][1]],
                   g1['n_w1'][:H], g1['n_w1'][H:],
                   rsh(g1['n_b1']), g1['n_w2'], rsh(g1['n_b2']),
                   cw1a1, cw1b1, rsh(e1['c_b1']))
    gg = [_gather(rows[i], cols[i], ab, ab) for i in (0, 1)]
    tr = [_edge2(gg[i][0], gg[i][1], m1[i], cw1c1, e1['c_w2'],
                 rsh(e1['c_b2']), w3_1) for i in (0, 1)]
    qq = [_scatter(rows[i], tr[i], z_w) for i in (0, 1)]
    pos_pad, h_out = _final(pp1, [qq[0][0], qq[0][1], qq[1][0], qq[1][1]],
                            h2, p['emb_out_w'], rsh(p['emb_out_b']))

    ea_pre = jnp.concatenate(
        [m1[0][:, H + 3:H + 6], m1[1][:, H + 3:H + 6]], axis=0)
    return h_out, pos_pad[:, :3], ea_pre
